# baseline (speedup 1.0000x reference)
import functools
import jax
import jax.numpy as jnp
from jax.experimental import pallas as pl
from jax.experimental.pallas import tpu as pltpu

HIDDEN = 1024
N_HEADS = 16
N_KV_HEADS = 4
HEAD_DIM = 64
N_EXPERTS = 8
TOP_K = 2
MOE_FF = 1408
SHARED_FF = 2816
EPS = 1e-6
ROPE_BASE = 1000000.0
T = 2048

BT = 256


def _dot(a, b):
    return jax.lax.dot_general(a.astype(jnp.bfloat16), b.astype(jnp.bfloat16),
                               (((1,), (0,)), ((), ())),
                               preferred_element_type=jnp.float32)


def _dot_t(a, b):
    return jax.lax.dot_general(a.astype(jnp.bfloat16), b.astype(jnp.bfloat16),
                               (((1,), (1,)), ((), ())),
                               preferred_element_type=jnp.float32)


def _rms(x, scale):
    var = jnp.mean(jnp.square(x), axis=-1, keepdims=True)
    return x * jax.lax.rsqrt(var + EPS) * scale


def _rope_2d(pos, x, n_heads):
    half = HEAD_DIM // 2
    x3 = x.reshape(x.shape[0], n_heads, HEAD_DIM)
    inv_freq = jnp.exp(
        jnp.arange(0, half, dtype=jnp.int32).astype(jnp.float32)
        * (-jnp.log(ROPE_BASE) / half))
    freqs = pos[:, None] * inv_freq[None, :]
    cos = jnp.cos(freqs)[:, None, :]
    sin = jnp.sin(freqs)[:, None, :]
    x1 = x3[..., :half]
    x2 = x3[..., half:]
    r = jnp.concatenate([x1 * cos - x2 * sin, x2 * cos + x1 * sin], axis=-1)
    return r.reshape(x.shape[0], n_heads * HEAD_DIM)


def _k1_body(pos_ref, h_ref, wq_ref, bq_ref, wk_ref, bk_ref, wv_ref, bv_ref,
             ln1_ref, q_ref, k_ref, v_ref):
    h = _rms(h_ref[...], ln1_ref[...])
    pos = pos_ref[0, 0, :].astype(jnp.float32)
    q = _dot(h, wq_ref[...]) + bq_ref[...]
    k = _dot(h, wk_ref[...]) + bk_ref[...]
    v = _dot(h, wv_ref[...]) + bv_ref[...]
    q_ref[...] = _rope_2d(pos, q, N_HEADS)
    k_ref[...] = _rope_2d(pos, k, N_KV_HEADS)
    v_ref[...] = v


def _pre_attn(positions, hidden_states, Wq, bq, Wk, bk, Wv, bv, ln1):
    pos3 = positions.reshape(T // BT, 1, BT)
    return pl.pallas_call(
        _k1_body,
        grid=(T // BT,),
        in_specs=[
            pl.BlockSpec((1, 1, BT), lambda i: (i, 0, 0)),
            pl.BlockSpec((BT, HIDDEN), lambda i: (i, 0)),
            pl.BlockSpec((HIDDEN, N_HEADS * HEAD_DIM), lambda i: (0, 0)),
            pl.BlockSpec((1, N_HEADS * HEAD_DIM), lambda i: (0, 0)),
            pl.BlockSpec((HIDDEN, N_KV_HEADS * HEAD_DIM), lambda i: (0, 0)),
            pl.BlockSpec((1, N_KV_HEADS * HEAD_DIM), lambda i: (0, 0)),
            pl.BlockSpec((HIDDEN, N_KV_HEADS * HEAD_DIM), lambda i: (0, 0)),
            pl.BlockSpec((1, N_KV_HEADS * HEAD_DIM), lambda i: (0, 0)),
            pl.BlockSpec((1, HIDDEN), lambda i: (0, 0)),
        ],
        out_specs=[
            pl.BlockSpec((BT, N_HEADS * HEAD_DIM), lambda i: (i, 0)),
            pl.BlockSpec((BT, N_KV_HEADS * HEAD_DIM), lambda i: (i, 0)),
            pl.BlockSpec((BT, N_KV_HEADS * HEAD_DIM), lambda i: (i, 0)),
        ],
        out_shape=[
            jax.ShapeDtypeStruct((T, N_HEADS * HEAD_DIM), jnp.float32),
            jax.ShapeDtypeStruct((T, N_KV_HEADS * HEAD_DIM), jnp.float32),
            jax.ShapeDtypeStruct((T, N_KV_HEADS * HEAD_DIM), jnp.float32),
        ],
    )(pos3, hidden_states, Wq, bq.reshape(1, -1), Wk, bk.reshape(1, -1),
      Wv, bv.reshape(1, -1), ln1.reshape(1, -1))


CK = 512
NKJ = T // CK


def _k2_body(q_ref, k_ref, v_ref, o_ref, acc_ref, m_ref, l_ref):
    i = pl.program_id(1)
    j = pl.program_id(2)
    scale = HEAD_DIM ** -0.5

    @pl.when(j == 0)
    def _():
        m_ref[...] = jnp.full(m_ref.shape, -1e30, jnp.float32)
        l_ref[...] = jnp.zeros(l_ref.shape, jnp.float32)
        acc_ref[...] = jnp.zeros(acc_ref.shape, jnp.float32)

    @pl.when(j <= i // (CK // BT))
    def _():
        q = q_ref[0]
        k = k_ref[0]
        v = v_ref[0]
        s = _dot_t(q, k) * scale
        r = i * BT + jax.lax.broadcasted_iota(jnp.int32, s.shape, 0)
        c = j * CK + jax.lax.broadcasted_iota(jnp.int32, s.shape, 1)
        s = jnp.where(c <= r, s, jnp.float32(-1e9))
        m_prev = m_ref[...]
        m_cur = jnp.maximum(m_prev, jnp.max(s, axis=-1, keepdims=True))
        alpha = jnp.exp(m_prev - m_cur)
        p = jnp.exp(s - m_cur)
        l_ref[...] = l_ref[...] * alpha + jnp.sum(p, axis=-1, keepdims=True)
        acc_ref[...] = acc_ref[...] * alpha + _dot(p, v)
        m_ref[...] = m_cur

    @pl.when(j == NKJ - 1)
    def _():
        o_ref[0] = acc_ref[...] / l_ref[...]


def _attention(q, k, v):
    rep = N_HEADS // N_KV_HEADS
    rr = CK // BT

    def kv_idx(h, i, j):
        return (h // rep, jnp.minimum(j, i // rr), 0)

    return pl.pallas_call(
        _k2_body,
        grid=(N_HEADS, T // BT, NKJ),
        in_specs=[
            pl.BlockSpec((1, BT, HEAD_DIM), lambda h, i, j: (h, i, 0)),
            pl.BlockSpec((1, CK, HEAD_DIM), kv_idx),
            pl.BlockSpec((1, CK, HEAD_DIM), kv_idx),
        ],
        out_specs=pl.BlockSpec((1, BT, HEAD_DIM), lambda h, i, j: (h, i, 0)),
        out_shape=jax.ShapeDtypeStruct((N_HEADS, T, HEAD_DIM), jnp.float32),
        scratch_shapes=[
            pltpu.VMEM((BT, HEAD_DIM), jnp.float32),
            pltpu.VMEM((BT, 1), jnp.float32),
            pltpu.VMEM((BT, 1), jnp.float32),
        ],
        compiler_params=pltpu.CompilerParams(
            dimension_semantics=("parallel", "arbitrary", "arbitrary")),
    )(q, k, v)


def _k3_body(attn_ref, wo_ref, res_ref, ln2_ref, res2_ref, h2_ref):
    hidden = _dot(attn_ref[...], wo_ref[...]) + res_ref[...]
    res2_ref[...] = hidden
    h2_ref[...] = _rms(hidden, ln2_ref[...])


def _post_attn(attn, Wo, residual, ln2):
    return pl.pallas_call(
        _k3_body,
        grid=(T // BT,),
        in_specs=[
            pl.BlockSpec((BT, N_HEADS * HEAD_DIM), lambda i: (i, 0)),
            pl.BlockSpec((N_HEADS * HEAD_DIM, HIDDEN), lambda i: (0, 0)),
            pl.BlockSpec((BT, HIDDEN), lambda i: (i, 0)),
            pl.BlockSpec((1, HIDDEN), lambda i: (0, 0)),
        ],
        out_specs=[
            pl.BlockSpec((BT, HIDDEN), lambda i: (i, 0)),
            pl.BlockSpec((BT, HIDDEN), lambda i: (i, 0)),
        ],
        out_shape=[
            jax.ShapeDtypeStruct((T, HIDDEN), jnp.float32),
            jax.ShapeDtypeStruct((T, HIDDEN), jnp.float32),
        ],
    )(attn, Wo, residual, ln2.reshape(1, -1))


def _k4_body(h2_ref, wsg_ref, wsu_ref, wsd_ref, wse_ref, out_ref):
    h2 = h2_ref[...]
    g = _dot(h2, wsg_ref[...])
    u = _dot(h2, wsu_ref[...])
    y = _dot(g * jax.lax.logistic(g) * u, wsd_ref[...])
    gate = jax.lax.logistic(_dot(h2, wse_ref[...]))
    out_ref[...] = gate * y


def _shared_expert(h2, Wsg, Wsu, Wsd, Wse):
    return pl.pallas_call(
        _k4_body,
        grid=(T // BT,),
        in_specs=[
            pl.BlockSpec((BT, HIDDEN), lambda i: (i, 0)),
            pl.BlockSpec((HIDDEN, SHARED_FF), lambda i: (0, 0)),
            pl.BlockSpec((HIDDEN, SHARED_FF), lambda i: (0, 0)),
            pl.BlockSpec((SHARED_FF, HIDDEN), lambda i: (0, 0)),
            pl.BlockSpec((HIDDEN, 1), lambda i: (0, 0)),
        ],
        out_specs=pl.BlockSpec((BT, HIDDEN), lambda i: (i, 0)),
        out_shape=jax.ShapeDtypeStruct((T, HIDDEN), jnp.float32),
    )(h2, Wsg, Wsu, Wsd, Wse)


BLK = 256
NBLK = 24
NROWS = NBLK * BLK


def _dotf(a, b):
    return jax.lax.dot_general(a, b, (((1,), (0,)), ((), ())),
                               preferred_element_type=jnp.float32,
                               precision=jax.lax.Precision.HIGHEST)


def _k5_body(h2_ref, wr_ref, w_ref, dest_ref, src_ref, be_ref, nb_ref):
    logits = _dot(h2_ref[...], wr_ref[...])
    m = jnp.max(logits, axis=-1, keepdims=True)
    e = jnp.exp(logits - m)
    probs = e / jnp.sum(e, axis=-1, keepdims=True)
    lane = jax.lax.broadcasted_iota(jnp.int32, probs.shape, 1)
    m1 = jnp.max(probs, axis=-1, keepdims=True)
    i1 = jnp.min(jnp.where(probs == m1, lane, N_EXPERTS), axis=-1,
                 keepdims=True)
    oh1 = (lane == i1).astype(jnp.float32)
    p2 = jnp.where(lane == i1, -1.0, probs)
    m2 = jnp.max(p2, axis=-1, keepdims=True)
    i2 = jnp.min(jnp.where(p2 == m2, lane, N_EXPERTS), axis=-1, keepdims=True)
    oh2 = (lane == i2).astype(jnp.float32)
    denom = m1 + m2
    w_ref[...] = jnp.concatenate([m1 / denom, m2 / denom], axis=1)

    oh = oh1 + oh2
    CH = 256
    tri = (jax.lax.broadcasted_iota(jnp.int32, (CH, CH), 1)
           < jax.lax.broadcasted_iota(jnp.int32, (CH, CH), 0)).astype(
               jnp.float32)
    carry = jnp.zeros((1, N_EXPERTS), jnp.float32)
    ranks = []
    for c in range(T // CH):
        sub = oh[c * CH:(c + 1) * CH, :]
        ranks.append(_dotf(tri, sub) + carry)
        carry = carry + jnp.sum(sub, axis=0, keepdims=True)
    rank = jnp.concatenate(ranks, axis=0)
    counts = carry
    padded = jnp.ceil(counts / BLK) * BLK
    tri8 = (jax.lax.broadcasted_iota(jnp.int32, (N_EXPERTS, N_EXPERTS), 0)
            < jax.lax.broadcasted_iota(jnp.int32, (N_EXPERTS, N_EXPERTS),
                                       1)).astype(jnp.float32)
    off = _dotf(padded, tri8)
    val = off + rank
    d0 = jnp.sum(oh1 * val, axis=1, keepdims=True)
    d1 = jnp.sum(oh2 * val, axis=1, keepdims=True)
    dest_ref[...] = jnp.concatenate([d0, d1], axis=1).astype(jnp.int32)

    d0r = d0.reshape(1, T)
    d1r = d1.reshape(1, T)
    tok = jax.lax.broadcasted_iota(jnp.int32, (T, 1), 0).astype(jnp.float32)
    for cb in range(NBLK):
        r = (cb * BLK
             + jax.lax.broadcasted_iota(jnp.int32, (BLK, 1), 0)).astype(
                 jnp.float32)
        eq = ((r == d0r) | (r == d1r)).astype(jnp.float32)
        src_ref[pl.ds(cb * BLK, BLK), :] = _dotf(eq, tok).astype(jnp.int32)

    endb = (off + padded) * (1.0 / BLK)
    b_iota = jax.lax.broadcasted_iota(jnp.int32, (1, NBLK), 1).astype(
        jnp.float32)
    acc = jnp.zeros((1, NBLK), jnp.float32)
    for ei in range(N_EXPERTS):
        acc = acc + (b_iota >= endb[0, ei]).astype(jnp.float32)
    be_ref[...] = jnp.minimum(acc, N_EXPERTS - 1).astype(jnp.int32)
    nb_ref[...] = (jnp.sum(padded, axis=1, keepdims=True)
                   * (1.0 / BLK)).astype(jnp.int32)


def _router(h2, Wr):
    return pl.pallas_call(
        _k5_body,
        grid=(1,),
        in_specs=[
            pl.BlockSpec((T, HIDDEN), lambda i: (0, 0)),
            pl.BlockSpec((HIDDEN, N_EXPERTS), lambda i: (0, 0)),
        ],
        out_specs=[
            pl.BlockSpec((T, 2), lambda i: (0, 0)),
            pl.BlockSpec((T, 2), lambda i: (0, 0)),
            pl.BlockSpec((NROWS, 1), lambda i: (0, 0)),
            pl.BlockSpec((1, NBLK), lambda i: (0, 0)),
            pl.BlockSpec((1, 1), lambda i: (0, 0)),
        ],
        out_shape=[
            jax.ShapeDtypeStruct((T, 2), jnp.float32),
            jax.ShapeDtypeStruct((T, 2), jnp.int32),
            jax.ShapeDtypeStruct((NROWS, 1), jnp.int32),
            jax.ShapeDtypeStruct((1, NBLK), jnp.int32),
            jax.ShapeDtypeStruct((1, 1), jnp.int32),
        ],
    )(h2, Wr)


def _sc_gather(table, idx, n_rows):
    from jax.experimental.pallas import tpu_sc as plsc
    info = plsc.get_sparse_core_info()
    nw = info.num_cores * info.num_subcores
    b_per_w = n_rows // nw
    ch = 64
    n_ch = b_per_w // ch
    mesh = plsc.VectorSubcoreMesh(core_axis_name="c", subcore_axis_name="s")

    @functools.partial(
        pl.kernel, mesh=mesh,
        out_type=jax.ShapeDtypeStruct((n_rows, HIDDEN), jnp.float32),
        scratch_types=[
            pltpu.VMEM((ch,), jnp.int32),
            pltpu.VMEM((ch, HIDDEN), jnp.float32),
            pltpu.SemaphoreType.DMA,
        ],
    )
    def k(table_hbm, idx_hbm, out_hbm, idx_v, rows_v, sem):
        wid = (jax.lax.axis_index("s") * info.num_cores
               + jax.lax.axis_index("c"))
        base = wid * b_per_w
        for c in range(n_ch):
            off = base + c * ch
            pltpu.sync_copy(idx_hbm.at[pl.ds(off, ch)], idx_v)
            pltpu.async_copy(table_hbm.at[idx_v], rows_v, sem).wait()
            pltpu.sync_copy(rows_v, out_hbm.at[pl.ds(off, ch)])

    return k(table, idx)


def _k6_body(be_ref, nb_ref, xs_ref, weg_ref, weu_ref, wed_ref, ys_ref):
    b = pl.program_id(0)

    @pl.when(b < nb_ref[0])
    def _():
        x = xs_ref[...]
        g = _dot(x, weg_ref[0])
        u = _dot(x, weu_ref[0])
        ys_ref[...] = _dot(g * jax.lax.logistic(g) * u, wed_ref[0])


def _grouped_moe(be, nb, xs, Weg, Weu, Wed):
    grid_spec = pltpu.PrefetchScalarGridSpec(
        num_scalar_prefetch=2,
        grid=(NBLK,),
        in_specs=[
            pl.BlockSpec((BLK, HIDDEN), lambda b, be_r, nb_r: (b, 0)),
            pl.BlockSpec((1, HIDDEN, MOE_FF),
                         lambda b, be_r, nb_r: (be_r[b], 0, 0)),
            pl.BlockSpec((1, HIDDEN, MOE_FF),
                         lambda b, be_r, nb_r: (be_r[b], 0, 0)),
            pl.BlockSpec((1, MOE_FF, HIDDEN),
                         lambda b, be_r, nb_r: (be_r[b], 0, 0)),
        ],
        out_specs=pl.BlockSpec((BLK, HIDDEN), lambda b, be_r, nb_r: (b, 0)),
    )
    return pl.pallas_call(
        _k6_body,
        grid_spec=grid_spec,
        out_shape=jax.ShapeDtypeStruct((NROWS, HIDDEN), jnp.float32),
    )(be.reshape(-1), nb.reshape(-1), xs, Weg, Weu, Wed)


def _k7_body(res2_ref, sh_ref, g0_ref, g1_ref, w_ref, out_ref):
    w = w_ref[...]
    out_ref[...] = (res2_ref[...] + sh_ref[...]
                    + w[:, 0:1] * g0_ref[...] + w[:, 1:2] * g1_ref[...])


def _final(res2, shared, g0, g1, w):
    return pl.pallas_call(
        _k7_body,
        grid=(T // BT,),
        in_specs=[
            pl.BlockSpec((BT, HIDDEN), lambda i: (i, 0)),
            pl.BlockSpec((BT, HIDDEN), lambda i: (i, 0)),
            pl.BlockSpec((BT, HIDDEN), lambda i: (i, 0)),
            pl.BlockSpec((BT, HIDDEN), lambda i: (i, 0)),
            pl.BlockSpec((BT, 2), lambda i: (i, 0)),
        ],
        out_specs=pl.BlockSpec((BT, HIDDEN), lambda i: (i, 0)),
        out_shape=jax.ShapeDtypeStruct((T, HIDDEN), jnp.float32),
        compiler_params=pltpu.CompilerParams(
            dimension_semantics=("parallel",)),
    )(res2, shared, g0, g1, w)


@jax.jit
def kernel(positions, hidden_states, Wq, bq, Wk, bk, Wv, bv, Wo, ln1, ln2,
           Wr, Weg, Weu, Wed, Wsg, Wsu, Wsd, Wse):
    q, k, v = _pre_attn(positions, hidden_states, Wq, bq, Wk, bk, Wv, bv, ln1)
    q3 = q.reshape(T, N_HEADS, HEAD_DIM).transpose(1, 0, 2)
    k3 = k.reshape(T, N_KV_HEADS, HEAD_DIM).transpose(1, 0, 2)
    v3 = v.reshape(T, N_KV_HEADS, HEAD_DIM).transpose(1, 0, 2)
    attn3 = _attention(q3, k3, v3)
    attn = attn3.transpose(1, 0, 2).reshape(T, N_HEADS * HEAD_DIM)
    res2, h2 = _post_attn(attn, Wo, hidden_states, ln2)
    w, dest, src, be, nb = _router(h2, Wr)
    xs = _sc_gather(h2, src.reshape(NROWS), NROWS)
    shared = _shared_expert(h2, Wsg, Wsu, Wsd, Wse)
    ys = _grouped_moe(be, nb, xs, Weg, Weu, Wed)
    g = _sc_gather(ys, dest.T.reshape(2 * T), 2 * T)
    return _final(res2, shared, g[:T], g[T:], w)


# speedup vs baseline: 1.1153x; 1.1153x over previous
import functools
import jax
import jax.numpy as jnp
from jax.experimental import pallas as pl
from jax.experimental.pallas import tpu as pltpu

HIDDEN = 1024
N_HEADS = 16
N_KV_HEADS = 4
HEAD_DIM = 64
N_EXPERTS = 8
TOP_K = 2
MOE_FF = 1408
SHARED_FF = 2816
EPS = 1e-6
ROPE_BASE = 1000000.0
T = 2048

BT = 256


def _dot(a, b):
    return jax.lax.dot_general(a.astype(jnp.bfloat16), b.astype(jnp.bfloat16),
                               (((1,), (0,)), ((), ())),
                               preferred_element_type=jnp.float32)


def _dot_t(a, b):
    return jax.lax.dot_general(a.astype(jnp.bfloat16), b.astype(jnp.bfloat16),
                               (((1,), (1,)), ((), ())),
                               preferred_element_type=jnp.float32)


def _rms(x, scale):
    var = jnp.mean(jnp.square(x), axis=-1, keepdims=True)
    return x * jax.lax.rsqrt(var + EPS) * scale


def _rope_2d(pos, x, n_heads):
    half = HEAD_DIM // 2
    x3 = x.reshape(x.shape[0], n_heads, HEAD_DIM)
    inv_freq = jnp.exp(
        jnp.arange(0, half, dtype=jnp.int32).astype(jnp.float32)
        * (-jnp.log(ROPE_BASE) / half))
    freqs = pos[:, None] * inv_freq[None, :]
    cos = jnp.cos(freqs)[:, None, :]
    sin = jnp.sin(freqs)[:, None, :]
    x1 = x3[..., :half]
    x2 = x3[..., half:]
    r = jnp.concatenate([x1 * cos - x2 * sin, x2 * cos + x1 * sin], axis=-1)
    return r.reshape(x.shape[0], n_heads * HEAD_DIM)


def _k1_body(pos_ref, h_ref, wq_ref, bq_ref, wk_ref, bk_ref, wv_ref, bv_ref,
             ln1_ref, q_ref, k_ref, v_ref):
    h = _rms(h_ref[...], ln1_ref[...])
    pos = pos_ref[0, 0, :].astype(jnp.float32)
    q = _dot(h, wq_ref[...]) + bq_ref[...]
    k = _dot(h, wk_ref[...]) + bk_ref[...]
    v = _dot(h, wv_ref[...]) + bv_ref[...]
    q_ref[...] = _rope_2d(pos, q, N_HEADS)
    k_ref[...] = _rope_2d(pos, k, N_KV_HEADS)
    v_ref[...] = v


def _pre_attn(positions, hidden_states, Wq, bq, Wk, bk, Wv, bv, ln1):
    pos3 = positions.reshape(T // BT, 1, BT)
    return pl.pallas_call(
        _k1_body,
        grid=(T // BT,),
        in_specs=[
            pl.BlockSpec((1, 1, BT), lambda i: (i, 0, 0)),
            pl.BlockSpec((BT, HIDDEN), lambda i: (i, 0)),
            pl.BlockSpec((HIDDEN, N_HEADS * HEAD_DIM), lambda i: (0, 0)),
            pl.BlockSpec((1, N_HEADS * HEAD_DIM), lambda i: (0, 0)),
            pl.BlockSpec((HIDDEN, N_KV_HEADS * HEAD_DIM), lambda i: (0, 0)),
            pl.BlockSpec((1, N_KV_HEADS * HEAD_DIM), lambda i: (0, 0)),
            pl.BlockSpec((HIDDEN, N_KV_HEADS * HEAD_DIM), lambda i: (0, 0)),
            pl.BlockSpec((1, N_KV_HEADS * HEAD_DIM), lambda i: (0, 0)),
            pl.BlockSpec((1, HIDDEN), lambda i: (0, 0)),
        ],
        out_specs=[
            pl.BlockSpec((BT, N_HEADS * HEAD_DIM), lambda i: (i, 0)),
            pl.BlockSpec((BT, N_KV_HEADS * HEAD_DIM), lambda i: (i, 0)),
            pl.BlockSpec((BT, N_KV_HEADS * HEAD_DIM), lambda i: (i, 0)),
        ],
        out_shape=[
            jax.ShapeDtypeStruct((T, N_HEADS * HEAD_DIM), jnp.float32),
            jax.ShapeDtypeStruct((T, N_KV_HEADS * HEAD_DIM), jnp.float32),
            jax.ShapeDtypeStruct((T, N_KV_HEADS * HEAD_DIM), jnp.float32),
        ],
    )(pos3, hidden_states, Wq, bq.reshape(1, -1), Wk, bk.reshape(1, -1),
      Wv, bv.reshape(1, -1), ln1.reshape(1, -1))


CK = 512
NKJ = T // CK


def _k2_body(q_ref, k_ref, v_ref, o_ref, acc_ref, m_ref, l_ref):
    i = pl.program_id(1)
    j = pl.program_id(2)
    scale = HEAD_DIM ** -0.5

    @pl.when(j == 0)
    def _():
        m_ref[...] = jnp.full(m_ref.shape, -1e30, jnp.float32)
        l_ref[...] = jnp.zeros(l_ref.shape, jnp.float32)
        acc_ref[...] = jnp.zeros(acc_ref.shape, jnp.float32)

    @pl.when(j <= i // (CK // BT))
    def _():
        q = q_ref[0]
        k = k_ref[0]
        v = v_ref[0]
        s = _dot_t(q, k) * scale
        r = i * BT + jax.lax.broadcasted_iota(jnp.int32, s.shape, 0)
        c = j * CK + jax.lax.broadcasted_iota(jnp.int32, s.shape, 1)
        s = jnp.where(c <= r, s, jnp.float32(-1e9))
        m_prev = m_ref[...]
        m_cur = jnp.maximum(m_prev, jnp.max(s, axis=-1, keepdims=True))
        alpha = jnp.exp(m_prev - m_cur)
        p = jnp.exp(s - m_cur)
        l_ref[...] = l_ref[...] * alpha + jnp.sum(p, axis=-1, keepdims=True)
        acc_ref[...] = acc_ref[...] * alpha + _dot(p, v)
        m_ref[...] = m_cur

    @pl.when(j == NKJ - 1)
    def _():
        o_ref[0] = acc_ref[...] / l_ref[...]


def _attention(q, k, v):
    rep = N_HEADS // N_KV_HEADS
    rr = CK // BT

    def kv_idx(h, i, j):
        return (h // rep, jnp.minimum(j, i // rr), 0)

    return pl.pallas_call(
        _k2_body,
        grid=(N_HEADS, T // BT, NKJ),
        in_specs=[
            pl.BlockSpec((1, BT, HEAD_DIM), lambda h, i, j: (h, i, 0)),
            pl.BlockSpec((1, CK, HEAD_DIM), kv_idx),
            pl.BlockSpec((1, CK, HEAD_DIM), kv_idx),
        ],
        out_specs=pl.BlockSpec((1, BT, HEAD_DIM), lambda h, i, j: (h, i, 0)),
        out_shape=jax.ShapeDtypeStruct((N_HEADS, T, HEAD_DIM), jnp.float32),
        scratch_shapes=[
            pltpu.VMEM((BT, HEAD_DIM), jnp.float32),
            pltpu.VMEM((BT, 1), jnp.float32),
            pltpu.VMEM((BT, 1), jnp.float32),
        ],
        compiler_params=pltpu.CompilerParams(
            dimension_semantics=("parallel", "arbitrary", "arbitrary")),
    )(q, k, v)


def _k3_body(attn_ref, wo_ref, res_ref, ln2_ref, res2_ref, h2_ref, h2b_ref):
    hidden = _dot(attn_ref[...], wo_ref[...]) + res_ref[...]
    res2_ref[...] = hidden
    h2 = _rms(hidden, ln2_ref[...])
    h2_ref[...] = h2
    h2b_ref[...] = h2.astype(jnp.bfloat16)


def _post_attn(attn, Wo, residual, ln2):
    return pl.pallas_call(
        _k3_body,
        grid=(T // BT,),
        in_specs=[
            pl.BlockSpec((BT, N_HEADS * HEAD_DIM), lambda i: (i, 0)),
            pl.BlockSpec((N_HEADS * HEAD_DIM, HIDDEN), lambda i: (0, 0)),
            pl.BlockSpec((BT, HIDDEN), lambda i: (i, 0)),
            pl.BlockSpec((1, HIDDEN), lambda i: (0, 0)),
        ],
        out_specs=[
            pl.BlockSpec((BT, HIDDEN), lambda i: (i, 0)),
            pl.BlockSpec((BT, HIDDEN), lambda i: (i, 0)),
            pl.BlockSpec((BT, HIDDEN), lambda i: (i, 0)),
        ],
        out_shape=[
            jax.ShapeDtypeStruct((T, HIDDEN), jnp.float32),
            jax.ShapeDtypeStruct((T, HIDDEN), jnp.float32),
            jax.ShapeDtypeStruct((T, HIDDEN), jnp.bfloat16),
        ],
    )(attn, Wo, residual, ln2.reshape(1, -1))


def _k4_body(h2_ref, wsg_ref, wsu_ref, wsd_ref, wse_ref, out_ref):
    h2 = h2_ref[...]
    g = _dot(h2, wsg_ref[...])
    u = _dot(h2, wsu_ref[...])
    y = _dot(g * jax.lax.logistic(g) * u, wsd_ref[...])
    gate = jax.lax.logistic(_dot(h2, wse_ref[...]))
    out_ref[...] = gate * y


def _shared_expert(h2, Wsg, Wsu, Wsd, Wse):
    return pl.pallas_call(
        _k4_body,
        grid=(T // BT,),
        in_specs=[
            pl.BlockSpec((BT, HIDDEN), lambda i: (i, 0)),
            pl.BlockSpec((HIDDEN, SHARED_FF), lambda i: (0, 0)),
            pl.BlockSpec((HIDDEN, SHARED_FF), lambda i: (0, 0)),
            pl.BlockSpec((SHARED_FF, HIDDEN), lambda i: (0, 0)),
            pl.BlockSpec((HIDDEN, 1), lambda i: (0, 0)),
        ],
        out_specs=pl.BlockSpec((BT, HIDDEN), lambda i: (i, 0)),
        out_shape=jax.ShapeDtypeStruct((T, HIDDEN), jnp.float32),
    )(h2, Wsg, Wsu, Wsd, Wse)


BLK = 256
NBLK = 24
NROWS = NBLK * BLK


def _dotf(a, b):
    return jax.lax.dot_general(a, b, (((1,), (0,)), ((), ())),
                               preferred_element_type=jnp.float32,
                               precision=jax.lax.Precision.HIGHEST)


def _k5_body(h2_ref, wr_ref, w_ref, dest_ref, src_ref, be_ref, nb_ref):
    logits = _dot(h2_ref[...], wr_ref[...])
    m = jnp.max(logits, axis=-1, keepdims=True)
    e = jnp.exp(logits - m)
    probs = e / jnp.sum(e, axis=-1, keepdims=True)
    lane = jax.lax.broadcasted_iota(jnp.int32, probs.shape, 1)
    m1 = jnp.max(probs, axis=-1, keepdims=True)
    i1 = jnp.min(jnp.where(probs == m1, lane, N_EXPERTS), axis=-1,
                 keepdims=True)
    oh1 = (lane == i1).astype(jnp.float32)
    p2 = jnp.where(lane == i1, -1.0, probs)
    m2 = jnp.max(p2, axis=-1, keepdims=True)
    i2 = jnp.min(jnp.where(p2 == m2, lane, N_EXPERTS), axis=-1, keepdims=True)
    oh2 = (lane == i2).astype(jnp.float32)
    denom = m1 + m2
    w_ref[...] = jnp.concatenate([m1 / denom, m2 / denom], axis=1)

    oh = oh1 + oh2
    CH = 256
    tri = (jax.lax.broadcasted_iota(jnp.int32, (CH, CH), 1)
           < jax.lax.broadcasted_iota(jnp.int32, (CH, CH), 0)).astype(
               jnp.float32)
    carry = jnp.zeros((1, N_EXPERTS), jnp.float32)
    ranks = []
    for c in range(T // CH):
        sub = oh[c * CH:(c + 1) * CH, :]
        ranks.append(_dotf(tri, sub) + carry)
        carry = carry + jnp.sum(sub, axis=0, keepdims=True)
    rank = jnp.concatenate(ranks, axis=0)
    counts = carry
    padded = jnp.ceil(counts / BLK) * BLK
    tri8 = (jax.lax.broadcasted_iota(jnp.int32, (N_EXPERTS, N_EXPERTS), 0)
            < jax.lax.broadcasted_iota(jnp.int32, (N_EXPERTS, N_EXPERTS),
                                       1)).astype(jnp.float32)
    off = _dotf(padded, tri8)
    val = off + rank
    d0 = jnp.sum(oh1 * val, axis=1, keepdims=True)
    d1 = jnp.sum(oh2 * val, axis=1, keepdims=True)
    dest_ref[...] = jnp.concatenate([d0, d1], axis=1).astype(jnp.int32)

    d0r = d0.reshape(1, T)
    d1r = d1.reshape(1, T)
    tok = jax.lax.broadcasted_iota(jnp.int32, (T, 1), 0).astype(jnp.float32)
    tokones = jnp.concatenate([tok, jnp.ones_like(tok)], axis=1)

    def _src_block(cb, carry):
        r = (cb * BLK + jax.lax.broadcasted_iota(
            jnp.int32, (BLK, 1), 0)).astype(jnp.float32)
        eq = ((r == d0r) | (r == d1r)).astype(jnp.float32)
        mh = _dotf(eq, tokones)
        matched = mh[:, 0:1]
        hit = mh[:, 1:2]
        filler = r - jnp.floor(r * (1.0 / T)) * T
        src = matched + (1.0 - hit) * filler
        src_ref[pl.ds(cb * BLK, BLK), :] = src.astype(jnp.int32)
        return carry

    jax.lax.fori_loop(0, NBLK, _src_block, 0)

    endb = (off + padded) * (1.0 / BLK)
    b_iota = jax.lax.broadcasted_iota(jnp.int32, (1, NBLK), 1).astype(
        jnp.float32)
    acc = jnp.zeros((1, NBLK), jnp.float32)
    for ei in range(N_EXPERTS):
        acc = acc + (b_iota >= endb[0, ei]).astype(jnp.float32)
    be_ref[...] = jnp.minimum(acc, N_EXPERTS - 1).astype(jnp.int32)
    nb_ref[...] = (jnp.sum(padded, axis=1, keepdims=True)
                   * (1.0 / BLK)).astype(jnp.int32)


def _router(h2, Wr):
    return pl.pallas_call(
        _k5_body,
        grid=(1,),
        in_specs=[
            pl.BlockSpec((T, HIDDEN), lambda i: (0, 0)),
            pl.BlockSpec((HIDDEN, N_EXPERTS), lambda i: (0, 0)),
        ],
        out_specs=[
            pl.BlockSpec((T, 2), lambda i: (0, 0)),
            pl.BlockSpec((T, 2), lambda i: (0, 0)),
            pl.BlockSpec((NROWS, 1), lambda i: (0, 0)),
            pl.BlockSpec((1, NBLK), lambda i: (0, 0)),
            pl.BlockSpec((1, 1), lambda i: (0, 0)),
        ],
        out_shape=[
            jax.ShapeDtypeStruct((T, 2), jnp.float32),
            jax.ShapeDtypeStruct((T, 2), jnp.int32),
            jax.ShapeDtypeStruct((NROWS, 1), jnp.int32),
            jax.ShapeDtypeStruct((1, NBLK), jnp.int32),
            jax.ShapeDtypeStruct((1, 1), jnp.int32),
        ],
    )(h2, Wr)


def _sc_gather(table, idx, n_rows, dtype):
    from jax.experimental.pallas import tpu_sc as plsc
    info = plsc.get_sparse_core_info()
    nw = info.num_cores * info.num_subcores
    b_per_w = n_rows // nw
    ch = 64
    n_ch = b_per_w // ch
    mesh = plsc.VectorSubcoreMesh(core_axis_name="c", subcore_axis_name="s")

    @functools.partial(
        pl.kernel, mesh=mesh,
        out_type=jax.ShapeDtypeStruct((n_rows, HIDDEN), dtype),
        scratch_types=[
            pltpu.VMEM((ch,), jnp.int32),
            pltpu.VMEM((ch, HIDDEN), dtype),
            pltpu.SemaphoreType.DMA,
        ],
    )
    def k(table_hbm, idx_hbm, out_hbm, idx_v, rows_v, sem):
        wid = (jax.lax.axis_index("s") * info.num_cores
               + jax.lax.axis_index("c"))
        base = wid * b_per_w
        for c in range(n_ch):
            off = base + c * ch
            pltpu.sync_copy(idx_hbm.at[pl.ds(off, ch)], idx_v)
            pltpu.async_copy(table_hbm.at[idx_v], rows_v, sem).wait()
            pltpu.sync_copy(rows_v, out_hbm.at[pl.ds(off, ch)])

    return k(table, idx)


def _k6_body(be_ref, nb_ref, xs_ref, weg_ref, weu_ref, wed_ref, ys_ref):
    b = pl.program_id(0)

    @pl.when(b < nb_ref[0])
    def _():
        x = xs_ref[...]
        g = _dot(x, weg_ref[0])
        u = _dot(x, weu_ref[0])
        ys_ref[...] = _dot(g * jax.lax.logistic(g) * u, wed_ref[0])


def _grouped_moe(be, nb, xs, Weg, Weu, Wed):
    grid_spec = pltpu.PrefetchScalarGridSpec(
        num_scalar_prefetch=2,
        grid=(NBLK,),
        in_specs=[
            pl.BlockSpec((BLK, HIDDEN), lambda b, be_r, nb_r: (b, 0)),
            pl.BlockSpec((1, HIDDEN, MOE_FF),
                         lambda b, be_r, nb_r: (be_r[b], 0, 0)),
            pl.BlockSpec((1, HIDDEN, MOE_FF),
                         lambda b, be_r, nb_r: (be_r[b], 0, 0)),
            pl.BlockSpec((1, MOE_FF, HIDDEN),
                         lambda b, be_r, nb_r: (be_r[b], 0, 0)),
        ],
        out_specs=pl.BlockSpec((BLK, HIDDEN), lambda b, be_r, nb_r: (b, 0)),
    )
    return pl.pallas_call(
        _k6_body,
        grid_spec=grid_spec,
        out_shape=jax.ShapeDtypeStruct((NROWS, HIDDEN), jnp.float32),
    )(be.reshape(-1), nb.reshape(-1), xs, Weg, Weu, Wed)


def _k7_body(res2_ref, sh_ref, g0_ref, g1_ref, w_ref, out_ref):
    w = w_ref[...]
    out_ref[...] = (res2_ref[...] + sh_ref[...]
                    + w[:, 0:1] * g0_ref[...] + w[:, 1:2] * g1_ref[...])


def _final(res2, shared, g0, g1, w):
    return pl.pallas_call(
        _k7_body,
        grid=(T // BT,),
        in_specs=[
            pl.BlockSpec((BT, HIDDEN), lambda i: (i, 0)),
            pl.BlockSpec((BT, HIDDEN), lambda i: (i, 0)),
            pl.BlockSpec((BT, HIDDEN), lambda i: (i, 0)),
            pl.BlockSpec((BT, HIDDEN), lambda i: (i, 0)),
            pl.BlockSpec((BT, 2), lambda i: (i, 0)),
        ],
        out_specs=pl.BlockSpec((BT, HIDDEN), lambda i: (i, 0)),
        out_shape=jax.ShapeDtypeStruct((T, HIDDEN), jnp.float32),
        compiler_params=pltpu.CompilerParams(
            dimension_semantics=("parallel",)),
    )(res2, shared, g0, g1, w)


@jax.jit
def kernel(positions, hidden_states, Wq, bq, Wk, bk, Wv, bv, Wo, ln1, ln2,
           Wr, Weg, Weu, Wed, Wsg, Wsu, Wsd, Wse):
    q, k, v = _pre_attn(positions, hidden_states, Wq, bq, Wk, bk, Wv, bv, ln1)
    q3 = q.reshape(T, N_HEADS, HEAD_DIM).transpose(1, 0, 2)
    k3 = k.reshape(T, N_KV_HEADS, HEAD_DIM).transpose(1, 0, 2)
    v3 = v.reshape(T, N_KV_HEADS, HEAD_DIM).transpose(1, 0, 2)
    attn3 = _attention(q3, k3, v3)
    attn = attn3.transpose(1, 0, 2).reshape(T, N_HEADS * HEAD_DIM)
    res2, h2, h2b = _post_attn(attn, Wo, hidden_states, ln2)
    w, dest, src, be, nb = _router(h2, Wr)
    xs = _sc_gather(h2, src.reshape(NROWS), NROWS, jnp.float32)
    shared = _shared_expert(h2, Wsg, Wsu, Wsd, Wse)
    ys = _grouped_moe(be, nb, xs, Weg, Weu, Wed)
    g = _sc_gather(ys, dest.T.reshape(2 * T), 2 * T, jnp.float32)
    return _final(res2, shared, g[:T], g[T:], w)


# speedup vs baseline: 1.2640x; 1.1334x over previous
import functools
import jax
import jax.numpy as jnp
from jax.experimental import pallas as pl
from jax.experimental.pallas import tpu as pltpu

HIDDEN = 1024
N_HEADS = 16
N_KV_HEADS = 4
HEAD_DIM = 64
N_EXPERTS = 8
TOP_K = 2
MOE_FF = 1408
SHARED_FF = 2816
EPS = 1e-6
ROPE_BASE = 1000000.0
T = 2048

BT = 256


def _dot(a, b):
    return jax.lax.dot_general(a.astype(jnp.bfloat16), b.astype(jnp.bfloat16),
                               (((1,), (0,)), ((), ())),
                               preferred_element_type=jnp.float32)


def _dot_t(a, b):
    return jax.lax.dot_general(a.astype(jnp.bfloat16), b.astype(jnp.bfloat16),
                               (((1,), (1,)), ((), ())),
                               preferred_element_type=jnp.float32)


def _rms(x, scale):
    var = jnp.mean(jnp.square(x), axis=-1, keepdims=True)
    return x * jax.lax.rsqrt(var + EPS) * scale


def _rope_2d(pos, x, n_heads):
    half = HEAD_DIM // 2
    x3 = x.reshape(x.shape[0], n_heads, HEAD_DIM)
    inv_freq = jnp.exp(
        jnp.arange(0, half, dtype=jnp.int32).astype(jnp.float32)
        * (-jnp.log(ROPE_BASE) / half))
    freqs = pos[:, None] * inv_freq[None, :]
    cos = jnp.cos(freqs)[:, None, :]
    sin = jnp.sin(freqs)[:, None, :]
    x1 = x3[..., :half]
    x2 = x3[..., half:]
    r = jnp.concatenate([x1 * cos - x2 * sin, x2 * cos + x1 * sin], axis=-1)
    return r.reshape(x.shape[0], n_heads * HEAD_DIM)


def _k1_body(pos_ref, h_ref, wq_ref, bq_ref, wk_ref, bk_ref, wv_ref, bv_ref,
             ln1_ref, q_ref, k_ref, v_ref):
    h = _rms(h_ref[...], ln1_ref[...])
    pos = pos_ref[0, 0, :].astype(jnp.float32)
    q = _dot(h, wq_ref[...]) + bq_ref[...]
    k = _dot(h, wk_ref[...]) + bk_ref[...]
    v = _dot(h, wv_ref[...]) + bv_ref[...]
    q_ref[...] = _rope_2d(pos, q, N_HEADS)
    k_ref[...] = _rope_2d(pos, k, N_KV_HEADS)
    v_ref[...] = v


def _pre_attn(positions, hidden_states, Wq, bq, Wk, bk, Wv, bv, ln1):
    pos3 = positions.reshape(T // BT, 1, BT)
    return pl.pallas_call(
        _k1_body,
        grid=(T // BT,),
        in_specs=[
            pl.BlockSpec((1, 1, BT), lambda i: (i, 0, 0)),
            pl.BlockSpec((BT, HIDDEN), lambda i: (i, 0)),
            pl.BlockSpec((HIDDEN, N_HEADS * HEAD_DIM), lambda i: (0, 0)),
            pl.BlockSpec((1, N_HEADS * HEAD_DIM), lambda i: (0, 0)),
            pl.BlockSpec((HIDDEN, N_KV_HEADS * HEAD_DIM), lambda i: (0, 0)),
            pl.BlockSpec((1, N_KV_HEADS * HEAD_DIM), lambda i: (0, 0)),
            pl.BlockSpec((HIDDEN, N_KV_HEADS * HEAD_DIM), lambda i: (0, 0)),
            pl.BlockSpec((1, N_KV_HEADS * HEAD_DIM), lambda i: (0, 0)),
            pl.BlockSpec((1, HIDDEN), lambda i: (0, 0)),
        ],
        out_specs=[
            pl.BlockSpec((BT, N_HEADS * HEAD_DIM), lambda i: (i, 0)),
            pl.BlockSpec((BT, N_KV_HEADS * HEAD_DIM), lambda i: (i, 0)),
            pl.BlockSpec((BT, N_KV_HEADS * HEAD_DIM), lambda i: (i, 0)),
        ],
        out_shape=[
            jax.ShapeDtypeStruct((T, N_HEADS * HEAD_DIM), jnp.float32),
            jax.ShapeDtypeStruct((T, N_KV_HEADS * HEAD_DIM), jnp.float32),
            jax.ShapeDtypeStruct((T, N_KV_HEADS * HEAD_DIM), jnp.float32),
        ],
    )(pos3, hidden_states, Wq, bq.reshape(1, -1), Wk, bk.reshape(1, -1),
      Wv, bv.reshape(1, -1), ln1.reshape(1, -1))


BQ = 512
CK = 512
NKJ = T // CK


def _k2_body(q_ref, k_ref, v_ref, o_ref, acc_ref, m_ref, l_ref):
    i = pl.program_id(1)
    j = pl.program_id(2)
    scale = HEAD_DIM ** -0.5

    @pl.when(j == 0)
    def _():
        m_ref[...] = jnp.full(m_ref.shape, -1e30, jnp.float32)
        l_ref[...] = jnp.zeros(l_ref.shape, jnp.float32)
        acc_ref[...] = jnp.zeros(acc_ref.shape, jnp.float32)

    @pl.when(j <= i // (CK // BQ))
    def _():
        q = q_ref[0]
        k = k_ref[0]
        v = v_ref[0]
        s = _dot_t(q, k) * scale
        r = i * BQ + jax.lax.broadcasted_iota(jnp.int32, s.shape, 0)
        c = j * CK + jax.lax.broadcasted_iota(jnp.int32, s.shape, 1)
        s = jnp.where(c <= r, s, jnp.float32(-1e9))
        m_prev = m_ref[...]
        m_cur = jnp.maximum(m_prev, jnp.max(s, axis=-1, keepdims=True))
        alpha = jnp.exp(m_prev - m_cur)
        p = jnp.exp(s - m_cur)
        l_ref[...] = l_ref[...] * alpha + jnp.sum(p, axis=-1, keepdims=True)
        acc_ref[...] = acc_ref[...] * alpha + _dot(p, v)
        m_ref[...] = m_cur

    @pl.when(j == NKJ - 1)
    def _():
        o_ref[0] = acc_ref[...] / l_ref[...]


def _attention(q, k, v):
    rep = N_HEADS // N_KV_HEADS
    rr = CK // BQ

    def kv_idx(h, i, j):
        return (h // rep, jnp.minimum(j, i // rr), 0)

    return pl.pallas_call(
        _k2_body,
        grid=(N_HEADS, T // BQ, NKJ),
        in_specs=[
            pl.BlockSpec((1, BQ, HEAD_DIM), lambda h, i, j: (h, i, 0)),
            pl.BlockSpec((1, CK, HEAD_DIM), kv_idx),
            pl.BlockSpec((1, CK, HEAD_DIM), kv_idx),
        ],
        out_specs=pl.BlockSpec((1, BQ, HEAD_DIM), lambda h, i, j: (h, i, 0)),
        out_shape=jax.ShapeDtypeStruct((N_HEADS, T, HEAD_DIM), jnp.float32),
        scratch_shapes=[
            pltpu.VMEM((BQ, HEAD_DIM), jnp.float32),
            pltpu.VMEM((BQ, 1), jnp.float32),
            pltpu.VMEM((BQ, 1), jnp.float32),
        ],
        compiler_params=pltpu.CompilerParams(
            dimension_semantics=("parallel", "arbitrary", "arbitrary")),
    )(q, k, v)


def _k3_body(attn_ref, wo_ref, res_ref, ln2_ref, res2_ref, h2_ref, h2b_ref):
    hidden = _dot(attn_ref[...], wo_ref[...]) + res_ref[...]
    res2_ref[...] = hidden
    h2 = _rms(hidden, ln2_ref[...])
    h2_ref[...] = h2
    h2b_ref[...] = h2.astype(jnp.bfloat16)


def _post_attn(attn, Wo, residual, ln2):
    return pl.pallas_call(
        _k3_body,
        grid=(T // BT,),
        in_specs=[
            pl.BlockSpec((BT, N_HEADS * HEAD_DIM), lambda i: (i, 0)),
            pl.BlockSpec((N_HEADS * HEAD_DIM, HIDDEN), lambda i: (0, 0)),
            pl.BlockSpec((BT, HIDDEN), lambda i: (i, 0)),
            pl.BlockSpec((1, HIDDEN), lambda i: (0, 0)),
        ],
        out_specs=[
            pl.BlockSpec((BT, HIDDEN), lambda i: (i, 0)),
            pl.BlockSpec((BT, HIDDEN), lambda i: (i, 0)),
            pl.BlockSpec((BT, HIDDEN), lambda i: (i, 0)),
        ],
        out_shape=[
            jax.ShapeDtypeStruct((T, HIDDEN), jnp.float32),
            jax.ShapeDtypeStruct((T, HIDDEN), jnp.float32),
            jax.ShapeDtypeStruct((T, HIDDEN), jnp.bfloat16),
        ],
    )(attn, Wo, residual, ln2.reshape(1, -1))


def _k4_body(h2_ref, wsg_ref, wsu_ref, wsd_ref, wse_ref, out_ref):
    h2 = h2_ref[...]
    g = _dot(h2, wsg_ref[...])
    u = _dot(h2, wsu_ref[...])
    y = _dot(g * jax.lax.logistic(g) * u, wsd_ref[...])
    gate = jax.lax.logistic(_dot(h2, wse_ref[...]))
    out_ref[...] = gate * y


def _shared_expert(h2, Wsg, Wsu, Wsd, Wse):
    return pl.pallas_call(
        _k4_body,
        grid=(T // BT,),
        in_specs=[
            pl.BlockSpec((BT, HIDDEN), lambda i: (i, 0)),
            pl.BlockSpec((HIDDEN, SHARED_FF), lambda i: (0, 0)),
            pl.BlockSpec((HIDDEN, SHARED_FF), lambda i: (0, 0)),
            pl.BlockSpec((SHARED_FF, HIDDEN), lambda i: (0, 0)),
            pl.BlockSpec((HIDDEN, 1), lambda i: (0, 0)),
        ],
        out_specs=pl.BlockSpec((BT, HIDDEN), lambda i: (i, 0)),
        out_shape=jax.ShapeDtypeStruct((T, HIDDEN), jnp.float32),
    )(h2, Wsg, Wsu, Wsd, Wse)


BLK = 256
NBLK = 24
NROWS = NBLK * BLK


def _dotf(a, b):
    return jax.lax.dot_general(a, b, (((1,), (0,)), ((), ())),
                               preferred_element_type=jnp.float32,
                               precision=jax.lax.Precision.HIGHEST)


def _k5_body(h2_ref, wr_ref, w_ref, dest_ref, src_ref, be_ref, nb_ref):
    logits = _dot(h2_ref[...], wr_ref[...])
    m = jnp.max(logits, axis=-1, keepdims=True)
    e = jnp.exp(logits - m)
    probs = e / jnp.sum(e, axis=-1, keepdims=True)
    lane = jax.lax.broadcasted_iota(jnp.int32, probs.shape, 1)
    m1 = jnp.max(probs, axis=-1, keepdims=True)
    i1 = jnp.min(jnp.where(probs == m1, lane, N_EXPERTS), axis=-1,
                 keepdims=True)
    oh1 = (lane == i1).astype(jnp.float32)
    p2 = jnp.where(lane == i1, -1.0, probs)
    m2 = jnp.max(p2, axis=-1, keepdims=True)
    i2 = jnp.min(jnp.where(p2 == m2, lane, N_EXPERTS), axis=-1, keepdims=True)
    oh2 = (lane == i2).astype(jnp.float32)
    denom = m1 + m2
    w_ref[...] = jnp.concatenate([m1 / denom, m2 / denom], axis=1)

    oh = oh1 + oh2
    CH = 256
    tri = (jax.lax.broadcasted_iota(jnp.int32, (CH, CH), 1)
           < jax.lax.broadcasted_iota(jnp.int32, (CH, CH), 0)).astype(
               jnp.float32)
    carry = jnp.zeros((1, N_EXPERTS), jnp.float32)
    ranks = []
    for c in range(T // CH):
        sub = oh[c * CH:(c + 1) * CH, :]
        ranks.append(_dotf(tri, sub) + carry)
        carry = carry + jnp.sum(sub, axis=0, keepdims=True)
    rank = jnp.concatenate(ranks, axis=0)
    counts = carry
    padded = jnp.ceil(counts / BLK) * BLK
    tri8 = (jax.lax.broadcasted_iota(jnp.int32, (N_EXPERTS, N_EXPERTS), 0)
            < jax.lax.broadcasted_iota(jnp.int32, (N_EXPERTS, N_EXPERTS),
                                       1)).astype(jnp.float32)
    off = _dotf(padded, tri8)
    val = off + rank
    d0 = jnp.sum(oh1 * val, axis=1, keepdims=True)
    d1 = jnp.sum(oh2 * val, axis=1, keepdims=True)
    dest_ref[...] = jnp.concatenate([d0, d1], axis=1).astype(jnp.int32)

    d0r = d0.reshape(1, T)
    d1r = d1.reshape(1, T)
    tok = jax.lax.broadcasted_iota(jnp.int32, (T, 1), 0).astype(jnp.float32)
    tokones = jnp.concatenate([tok, jnp.ones_like(tok)], axis=1)

    def _src_block(cb, carry):
        r = (cb * BLK + jax.lax.broadcasted_iota(
            jnp.int32, (BLK, 1), 0)).astype(jnp.float32)
        eq = ((r == d0r) | (r == d1r)).astype(jnp.float32)
        mh = _dotf(eq, tokones)
        matched = mh[:, 0:1]
        hit = mh[:, 1:2]
        filler = r - jnp.floor(r * (1.0 / T)) * T
        src = matched + (1.0 - hit) * filler
        src_ref[pl.ds(cb * BLK, BLK), :] = src.astype(jnp.int32)
        return carry

    jax.lax.fori_loop(0, NBLK, _src_block, 0)

    endb = (off + padded) * (1.0 / BLK)
    b_iota = jax.lax.broadcasted_iota(jnp.int32, (1, NBLK), 1).astype(
        jnp.float32)
    acc = jnp.zeros((1, NBLK), jnp.float32)
    for ei in range(N_EXPERTS):
        acc = acc + (b_iota >= endb[0, ei]).astype(jnp.float32)
    be_ref[...] = jnp.minimum(acc, N_EXPERTS - 1).astype(jnp.int32)
    nb_ref[...] = (jnp.sum(padded, axis=1, keepdims=True)
                   * (1.0 / BLK)).astype(jnp.int32)


def _router(h2, Wr):
    return pl.pallas_call(
        _k5_body,
        grid=(1,),
        in_specs=[
            pl.BlockSpec((T, HIDDEN), lambda i: (0, 0)),
            pl.BlockSpec((HIDDEN, N_EXPERTS), lambda i: (0, 0)),
        ],
        out_specs=[
            pl.BlockSpec((T, 2), lambda i: (0, 0)),
            pl.BlockSpec((T, 2), lambda i: (0, 0)),
            pl.BlockSpec((NROWS, 1), lambda i: (0, 0)),
            pl.BlockSpec((1, NBLK), lambda i: (0, 0)),
            pl.BlockSpec((1, 1), lambda i: (0, 0)),
        ],
        out_shape=[
            jax.ShapeDtypeStruct((T, 2), jnp.float32),
            jax.ShapeDtypeStruct((T, 2), jnp.int32),
            jax.ShapeDtypeStruct((NROWS, 1), jnp.int32),
            jax.ShapeDtypeStruct((1, NBLK), jnp.int32),
            jax.ShapeDtypeStruct((1, 1), jnp.int32),
        ],
    )(h2, Wr)


def _sc_gather(table, idx, n_rows, dtype):
    from jax.experimental.pallas import tpu_sc as plsc
    info = plsc.get_sparse_core_info()
    nw = info.num_cores * info.num_subcores
    b_per_w = n_rows // nw
    ch = 64
    n_ch = b_per_w // ch
    mesh = plsc.VectorSubcoreMesh(core_axis_name="c", subcore_axis_name="s")

    @functools.partial(
        pl.kernel, mesh=mesh,
        out_type=jax.ShapeDtypeStruct((n_rows, HIDDEN), dtype),
        scratch_types=[
            pltpu.VMEM((ch,), jnp.int32),
            pltpu.VMEM((ch, HIDDEN), dtype),
            pltpu.SemaphoreType.DMA,
        ],
    )
    def k(table_hbm, idx_hbm, out_hbm, idx_v, rows_v, sem):
        wid = (jax.lax.axis_index("s") * info.num_cores
               + jax.lax.axis_index("c"))
        base = wid * b_per_w
        for c in range(n_ch):
            off = base + c * ch
            pltpu.sync_copy(idx_hbm.at[pl.ds(off, ch)], idx_v)
            pltpu.async_copy(table_hbm.at[idx_v], rows_v, sem).wait()
            pltpu.sync_copy(rows_v, out_hbm.at[pl.ds(off, ch)])

    return k(table, idx)


def _k6_body(be_ref, nb_ref, xs_ref, weg_ref, weu_ref, wed_ref, ys_ref):
    b = pl.program_id(0)

    @pl.when(b < nb_ref[0])
    def _():
        x = xs_ref[...]
        g = _dot(x, weg_ref[0])
        u = _dot(x, weu_ref[0])
        ys_ref[...] = _dot(g * jax.lax.logistic(g) * u, wed_ref[0])


def _grouped_moe(be, nb, xs, Weg, Weu, Wed):
    grid_spec = pltpu.PrefetchScalarGridSpec(
        num_scalar_prefetch=2,
        grid=(NBLK,),
        in_specs=[
            pl.BlockSpec((BLK, HIDDEN), lambda b, be_r, nb_r: (b, 0)),
            pl.BlockSpec((1, HIDDEN, MOE_FF),
                         lambda b, be_r, nb_r: (be_r[b], 0, 0)),
            pl.BlockSpec((1, HIDDEN, MOE_FF),
                         lambda b, be_r, nb_r: (be_r[b], 0, 0)),
            pl.BlockSpec((1, MOE_FF, HIDDEN),
                         lambda b, be_r, nb_r: (be_r[b], 0, 0)),
        ],
        out_specs=pl.BlockSpec((BLK, HIDDEN), lambda b, be_r, nb_r: (b, 0)),
    )
    return pl.pallas_call(
        _k6_body,
        grid_spec=grid_spec,
        out_shape=jax.ShapeDtypeStruct((NROWS, HIDDEN), jnp.float32),
    )(be.reshape(-1), nb.reshape(-1), xs, Weg, Weu, Wed)


def _k7_body(res2_ref, sh_ref, g0_ref, g1_ref, w_ref, out_ref):
    w = w_ref[...]
    out_ref[...] = (res2_ref[...] + sh_ref[...]
                    + w[:, 0:1] * g0_ref[...] + w[:, 1:2] * g1_ref[...])


def _final(res2, shared, g0, g1, w):
    return pl.pallas_call(
        _k7_body,
        grid=(T // BT,),
        in_specs=[
            pl.BlockSpec((BT, HIDDEN), lambda i: (i, 0)),
            pl.BlockSpec((BT, HIDDEN), lambda i: (i, 0)),
            pl.BlockSpec((BT, HIDDEN), lambda i: (i, 0)),
            pl.BlockSpec((BT, HIDDEN), lambda i: (i, 0)),
            pl.BlockSpec((BT, 2), lambda i: (i, 0)),
        ],
        out_specs=pl.BlockSpec((BT, HIDDEN), lambda i: (i, 0)),
        out_shape=jax.ShapeDtypeStruct((T, HIDDEN), jnp.float32),
        compiler_params=pltpu.CompilerParams(
            dimension_semantics=("parallel",)),
    )(res2, shared, g0, g1, w)


@jax.jit
def kernel(positions, hidden_states, Wq, bq, Wk, bk, Wv, bv, Wo, ln1, ln2,
           Wr, Weg, Weu, Wed, Wsg, Wsu, Wsd, Wse):
    q, k, v = _pre_attn(positions, hidden_states, Wq, bq, Wk, bk, Wv, bv, ln1)
    q3 = q.reshape(T, N_HEADS, HEAD_DIM).transpose(1, 0, 2)
    k3 = k.reshape(T, N_KV_HEADS, HEAD_DIM).transpose(1, 0, 2)
    v3 = v.reshape(T, N_KV_HEADS, HEAD_DIM).transpose(1, 0, 2)
    attn3 = _attention(q3, k3, v3)
    attn = attn3.transpose(1, 0, 2).reshape(T, N_HEADS * HEAD_DIM)
    res2, h2, h2b = _post_attn(attn, Wo, hidden_states, ln2)
    w, dest, src, be, nb = _router(h2, Wr)
    xs = _sc_gather(h2, src.reshape(NROWS), NROWS, jnp.float32)
    shared = _shared_expert(h2, Wsg, Wsu, Wsd, Wse)
    ys = _grouped_moe(be, nb, xs, Weg, Weu, Wed)
    g = _sc_gather(ys, dest.T.reshape(2 * T), 2 * T, jnp.float32)
    return _final(res2, shared, g[:T], g[T:], w)


# speedup vs baseline: 1.3873x; 1.0976x over previous
import functools
import jax
import jax.numpy as jnp
from jax.experimental import pallas as pl
from jax.experimental.pallas import tpu as pltpu

HIDDEN = 1024
N_HEADS = 16
N_KV_HEADS = 4
HEAD_DIM = 64
N_EXPERTS = 8
TOP_K = 2
MOE_FF = 1408
SHARED_FF = 2816
EPS = 1e-6
ROPE_BASE = 1000000.0
T = 2048

BT = 256


def _dot(a, b):
    return jax.lax.dot_general(a.astype(jnp.bfloat16), b.astype(jnp.bfloat16),
                               (((1,), (0,)), ((), ())),
                               preferred_element_type=jnp.float32)


def _dot_t(a, b):
    return jax.lax.dot_general(a.astype(jnp.bfloat16), b.astype(jnp.bfloat16),
                               (((1,), (1,)), ((), ())),
                               preferred_element_type=jnp.float32)


def _rms(x, scale):
    var = jnp.mean(jnp.square(x), axis=-1, keepdims=True)
    return x * jax.lax.rsqrt(var + EPS) * scale


def _rope_2d(pos, x, n_heads):
    half = HEAD_DIM // 2
    x3 = x.reshape(x.shape[0], n_heads, HEAD_DIM)
    inv_freq = jnp.exp(
        jnp.arange(0, half, dtype=jnp.int32).astype(jnp.float32)
        * (-jnp.log(ROPE_BASE) / half))
    freqs = pos[:, None] * inv_freq[None, :]
    cos = jnp.cos(freqs)[:, None, :]
    sin = jnp.sin(freqs)[:, None, :]
    x1 = x3[..., :half]
    x2 = x3[..., half:]
    r = jnp.concatenate([x1 * cos - x2 * sin, x2 * cos + x1 * sin], axis=-1)
    return r.reshape(x.shape[0], n_heads * HEAD_DIM)


def _k1_body(pos_ref, h_ref, wq_ref, bq_ref, wk_ref, bk_ref, wv_ref, bv_ref,
             ln1_ref, q_ref, k_ref, v_ref):
    h = _rms(h_ref[...], ln1_ref[...])
    pos = pos_ref[0, 0, :].astype(jnp.float32)
    q = _dot(h, wq_ref[...]) + bq_ref[...]
    k = _dot(h, wk_ref[...]) + bk_ref[...]
    v = _dot(h, wv_ref[...]) + bv_ref[...]
    q_ref[...] = _rope_2d(pos, q, N_HEADS)
    k_ref[...] = _rope_2d(pos, k, N_KV_HEADS)
    v_ref[...] = v


def _pre_attn(positions, hidden_states, Wq, bq, Wk, bk, Wv, bv, ln1):
    pos3 = positions.reshape(T // BT, 1, BT)
    return pl.pallas_call(
        _k1_body,
        grid=(T // BT,),
        in_specs=[
            pl.BlockSpec((1, 1, BT), lambda i: (i, 0, 0)),
            pl.BlockSpec((BT, HIDDEN), lambda i: (i, 0)),
            pl.BlockSpec((HIDDEN, N_HEADS * HEAD_DIM), lambda i: (0, 0)),
            pl.BlockSpec((1, N_HEADS * HEAD_DIM), lambda i: (0, 0)),
            pl.BlockSpec((HIDDEN, N_KV_HEADS * HEAD_DIM), lambda i: (0, 0)),
            pl.BlockSpec((1, N_KV_HEADS * HEAD_DIM), lambda i: (0, 0)),
            pl.BlockSpec((HIDDEN, N_KV_HEADS * HEAD_DIM), lambda i: (0, 0)),
            pl.BlockSpec((1, N_KV_HEADS * HEAD_DIM), lambda i: (0, 0)),
            pl.BlockSpec((1, HIDDEN), lambda i: (0, 0)),
        ],
        out_specs=[
            pl.BlockSpec((BT, N_HEADS * HEAD_DIM), lambda i: (i, 0)),
            pl.BlockSpec((BT, N_KV_HEADS * HEAD_DIM), lambda i: (i, 0)),
            pl.BlockSpec((BT, N_KV_HEADS * HEAD_DIM), lambda i: (i, 0)),
        ],
        out_shape=[
            jax.ShapeDtypeStruct((T, N_HEADS * HEAD_DIM), jnp.float32),
            jax.ShapeDtypeStruct((T, N_KV_HEADS * HEAD_DIM), jnp.float32),
            jax.ShapeDtypeStruct((T, N_KV_HEADS * HEAD_DIM), jnp.float32),
        ],
    )(pos3, hidden_states, Wq, bq.reshape(1, -1), Wk, bk.reshape(1, -1),
      Wv, bv.reshape(1, -1), ln1.reshape(1, -1))


BQ = 512
CK = 1024
NKJ = T // CK


def _k2_body(q_ref, k_ref, v_ref, o_ref, acc_ref, m_ref, l_ref):
    i = pl.program_id(1)
    j = pl.program_id(2)
    scale = HEAD_DIM ** -0.5

    @pl.when(j == 0)
    def _():
        m_ref[...] = jnp.full(m_ref.shape, -1e30, jnp.float32)
        l_ref[...] = jnp.zeros(l_ref.shape, jnp.float32)
        acc_ref[...] = jnp.zeros(acc_ref.shape, jnp.float32)

    @pl.when(j <= i // (CK // BQ))
    def _():
        q = q_ref[0]
        k = k_ref[0]
        v = v_ref[0]
        s = _dot_t(q, k) * scale
        r = i * BQ + jax.lax.broadcasted_iota(jnp.int32, s.shape, 0)
        c = j * CK + jax.lax.broadcasted_iota(jnp.int32, s.shape, 1)
        s = jnp.where(c <= r, s, jnp.float32(-1e9))
        m_prev = m_ref[...]
        m_cur = jnp.maximum(m_prev, jnp.max(s, axis=-1, keepdims=True))
        alpha = jnp.exp(m_prev - m_cur)
        p = jnp.exp(s - m_cur)
        l_ref[...] = l_ref[...] * alpha + jnp.sum(p, axis=-1, keepdims=True)
        acc_ref[...] = acc_ref[...] * alpha + _dot(p, v)
        m_ref[...] = m_cur

    @pl.when(j == NKJ - 1)
    def _():
        o_ref[0] = acc_ref[...] / l_ref[...]


def _attention(q, k, v):
    rep = N_HEADS // N_KV_HEADS
    rr = CK // BQ

    def kv_idx(h, i, j):
        return (h // rep, jnp.minimum(j, i // rr), 0)

    return pl.pallas_call(
        _k2_body,
        grid=(N_HEADS, T // BQ, NKJ),
        in_specs=[
            pl.BlockSpec((1, BQ, HEAD_DIM), lambda h, i, j: (h, i, 0)),
            pl.BlockSpec((1, CK, HEAD_DIM), kv_idx),
            pl.BlockSpec((1, CK, HEAD_DIM), kv_idx),
        ],
        out_specs=pl.BlockSpec((1, BQ, HEAD_DIM), lambda h, i, j: (h, i, 0)),
        out_shape=jax.ShapeDtypeStruct((N_HEADS, T, HEAD_DIM), jnp.float32),
        scratch_shapes=[
            pltpu.VMEM((BQ, HEAD_DIM), jnp.float32),
            pltpu.VMEM((BQ, 1), jnp.float32),
            pltpu.VMEM((BQ, 1), jnp.float32),
        ],
        compiler_params=pltpu.CompilerParams(
            dimension_semantics=("parallel", "arbitrary", "arbitrary")),
    )(q, k, v)


def _k3_body(attn_ref, wo_ref, res_ref, ln2_ref, res2_ref, h2_ref, h2b_ref):
    hidden = _dot(attn_ref[...], wo_ref[...]) + res_ref[...]
    res2_ref[...] = hidden
    h2 = _rms(hidden, ln2_ref[...])
    h2_ref[...] = h2
    h2b_ref[...] = h2.astype(jnp.bfloat16)


def _post_attn(attn, Wo, residual, ln2):
    return pl.pallas_call(
        _k3_body,
        grid=(T // BT,),
        in_specs=[
            pl.BlockSpec((BT, N_HEADS * HEAD_DIM), lambda i: (i, 0)),
            pl.BlockSpec((N_HEADS * HEAD_DIM, HIDDEN), lambda i: (0, 0)),
            pl.BlockSpec((BT, HIDDEN), lambda i: (i, 0)),
            pl.BlockSpec((1, HIDDEN), lambda i: (0, 0)),
        ],
        out_specs=[
            pl.BlockSpec((BT, HIDDEN), lambda i: (i, 0)),
            pl.BlockSpec((BT, HIDDEN), lambda i: (i, 0)),
            pl.BlockSpec((BT, HIDDEN), lambda i: (i, 0)),
        ],
        out_shape=[
            jax.ShapeDtypeStruct((T, HIDDEN), jnp.float32),
            jax.ShapeDtypeStruct((T, HIDDEN), jnp.float32),
            jax.ShapeDtypeStruct((T, HIDDEN), jnp.bfloat16),
        ],
    )(attn, Wo, residual, ln2.reshape(1, -1))


def _k4_body(h2_ref, wsg_ref, wsu_ref, wsd_ref, wse_ref, out_ref):
    h2 = h2_ref[...]
    g = _dot(h2, wsg_ref[...])
    u = _dot(h2, wsu_ref[...])
    y = _dot(g * jax.lax.logistic(g) * u, wsd_ref[...])
    gate = jax.lax.logistic(_dot(h2, wse_ref[...]))
    out_ref[...] = gate * y


def _shared_expert(h2, Wsg, Wsu, Wsd, Wse):
    return pl.pallas_call(
        _k4_body,
        grid=(T // BT,),
        in_specs=[
            pl.BlockSpec((BT, HIDDEN), lambda i: (i, 0)),
            pl.BlockSpec((HIDDEN, SHARED_FF), lambda i: (0, 0)),
            pl.BlockSpec((HIDDEN, SHARED_FF), lambda i: (0, 0)),
            pl.BlockSpec((SHARED_FF, HIDDEN), lambda i: (0, 0)),
            pl.BlockSpec((HIDDEN, 1), lambda i: (0, 0)),
        ],
        out_specs=pl.BlockSpec((BT, HIDDEN), lambda i: (i, 0)),
        out_shape=jax.ShapeDtypeStruct((T, HIDDEN), jnp.float32),
    )(h2, Wsg, Wsu, Wsd, Wse)


BLK = 256
NBLK = 24
NROWS = NBLK * BLK


def _dotf(a, b):
    return jax.lax.dot_general(a, b, (((1,), (0,)), ((), ())),
                               preferred_element_type=jnp.float32,
                               precision=jax.lax.Precision.HIGHEST)


def _k5_body(h2_ref, wr_ref, w_ref, dest_ref, src_ref, be_ref, nb_ref):
    logits = _dot(h2_ref[...], wr_ref[...])
    m = jnp.max(logits, axis=-1, keepdims=True)
    e = jnp.exp(logits - m)
    probs = e / jnp.sum(e, axis=-1, keepdims=True)
    lane = jax.lax.broadcasted_iota(jnp.int32, probs.shape, 1)
    m1 = jnp.max(probs, axis=-1, keepdims=True)
    i1 = jnp.min(jnp.where(probs == m1, lane, N_EXPERTS), axis=-1,
                 keepdims=True)
    oh1 = (lane == i1).astype(jnp.float32)
    p2 = jnp.where(lane == i1, -1.0, probs)
    m2 = jnp.max(p2, axis=-1, keepdims=True)
    i2 = jnp.min(jnp.where(p2 == m2, lane, N_EXPERTS), axis=-1, keepdims=True)
    oh2 = (lane == i2).astype(jnp.float32)
    denom = m1 + m2
    w_ref[...] = jnp.concatenate([m1 / denom, m2 / denom], axis=1)

    oh = oh1 + oh2
    CH = 256
    tri = (jax.lax.broadcasted_iota(jnp.int32, (CH, CH), 1)
           < jax.lax.broadcasted_iota(jnp.int32, (CH, CH), 0)).astype(
               jnp.float32)
    carry = jnp.zeros((1, N_EXPERTS), jnp.float32)
    ranks = []
    for c in range(T // CH):
        sub = oh[c * CH:(c + 1) * CH, :]
        ranks.append(_dotf(tri, sub) + carry)
        carry = carry + jnp.sum(sub, axis=0, keepdims=True)
    rank = jnp.concatenate(ranks, axis=0)
    counts = carry
    padded = jnp.ceil(counts / BLK) * BLK
    tri8 = (jax.lax.broadcasted_iota(jnp.int32, (N_EXPERTS, N_EXPERTS), 0)
            < jax.lax.broadcasted_iota(jnp.int32, (N_EXPERTS, N_EXPERTS),
                                       1)).astype(jnp.float32)
    off = _dotf(padded, tri8)
    val = off + rank
    d0 = jnp.sum(oh1 * val, axis=1, keepdims=True)
    d1 = jnp.sum(oh2 * val, axis=1, keepdims=True)
    dest_ref[...] = jnp.concatenate([d0, d1], axis=1).astype(jnp.int32)

    d0r = d0.reshape(1, T)
    d1r = d1.reshape(1, T)
    tok = jax.lax.broadcasted_iota(jnp.int32, (T, 1), 0).astype(jnp.float32)
    tokones = jnp.concatenate([tok, jnp.ones_like(tok)], axis=1)

    def _src_block(cb, carry):
        r = (cb * BLK + jax.lax.broadcasted_iota(
            jnp.int32, (BLK, 1), 0)).astype(jnp.float32)
        eq = ((r == d0r) | (r == d1r)).astype(jnp.float32)
        mh = _dotf(eq, tokones)
        matched = mh[:, 0:1]
        hit = mh[:, 1:2]
        filler = r - jnp.floor(r * (1.0 / T)) * T
        src = matched + (1.0 - hit) * filler
        src_ref[pl.ds(cb * BLK, BLK), :] = src.astype(jnp.int32)
        return carry

    jax.lax.fori_loop(0, NBLK, _src_block, 0)

    endb = (off + padded) * (1.0 / BLK)
    b_iota = jax.lax.broadcasted_iota(jnp.int32, (1, NBLK), 1).astype(
        jnp.float32)
    acc = jnp.zeros((1, NBLK), jnp.float32)
    for ei in range(N_EXPERTS):
        acc = acc + (b_iota >= endb[0, ei]).astype(jnp.float32)
    be_ref[...] = jnp.minimum(acc, N_EXPERTS - 1).astype(jnp.int32)
    nb_ref[...] = (jnp.sum(padded, axis=1, keepdims=True)
                   * (1.0 / BLK)).astype(jnp.int32)


def _router(h2, Wr):
    return pl.pallas_call(
        _k5_body,
        grid=(1,),
        in_specs=[
            pl.BlockSpec((T, HIDDEN), lambda i: (0, 0)),
            pl.BlockSpec((HIDDEN, N_EXPERTS), lambda i: (0, 0)),
        ],
        out_specs=[
            pl.BlockSpec((T, 2), lambda i: (0, 0)),
            pl.BlockSpec((T, 2), lambda i: (0, 0)),
            pl.BlockSpec((NROWS, 1), lambda i: (0, 0)),
            pl.BlockSpec((1, NBLK), lambda i: (0, 0)),
            pl.BlockSpec((1, 1), lambda i: (0, 0)),
        ],
        out_shape=[
            jax.ShapeDtypeStruct((T, 2), jnp.float32),
            jax.ShapeDtypeStruct((T, 2), jnp.int32),
            jax.ShapeDtypeStruct((NROWS, 1), jnp.int32),
            jax.ShapeDtypeStruct((1, NBLK), jnp.int32),
            jax.ShapeDtypeStruct((1, 1), jnp.int32),
        ],
    )(h2, Wr)


def _sc_gather(table, idx, n_rows, dtype):
    from jax.experimental.pallas import tpu_sc as plsc
    info = plsc.get_sparse_core_info()
    nw = info.num_cores * info.num_subcores
    b_per_w = n_rows // nw
    ch = 64
    n_ch = b_per_w // ch
    mesh = plsc.VectorSubcoreMesh(core_axis_name="c", subcore_axis_name="s")

    @functools.partial(
        pl.kernel, mesh=mesh,
        out_type=jax.ShapeDtypeStruct((n_rows, HIDDEN), dtype),
        scratch_types=[
            pltpu.VMEM((ch,), jnp.int32),
            pltpu.VMEM((ch, HIDDEN), dtype),
            pltpu.SemaphoreType.DMA,
        ],
    )
    def k(table_hbm, idx_hbm, out_hbm, idx_v, rows_v, sem):
        wid = (jax.lax.axis_index("s") * info.num_cores
               + jax.lax.axis_index("c"))
        base = wid * b_per_w
        for c in range(n_ch):
            off = base + c * ch
            pltpu.sync_copy(idx_hbm.at[pl.ds(off, ch)], idx_v)
            pltpu.async_copy(table_hbm.at[idx_v], rows_v, sem).wait()
            pltpu.sync_copy(rows_v, out_hbm.at[pl.ds(off, ch)])

    return k(table, idx)


def _k6_body(be_ref, nb_ref, xs_ref, weg_ref, weu_ref, wed_ref, ys_ref):
    b = pl.program_id(0)

    @pl.when(b < nb_ref[0])
    def _():
        x = xs_ref[...]
        g = _dot(x, weg_ref[0])
        u = _dot(x, weu_ref[0])
        ys_ref[...] = _dot(g * jax.lax.logistic(g) * u, wed_ref[0])


def _grouped_moe(be, nb, xs, Weg, Weu, Wed):
    grid_spec = pltpu.PrefetchScalarGridSpec(
        num_scalar_prefetch=2,
        grid=(NBLK,),
        in_specs=[
            pl.BlockSpec((BLK, HIDDEN), lambda b, be_r, nb_r: (b, 0)),
            pl.BlockSpec((1, HIDDEN, MOE_FF),
                         lambda b, be_r, nb_r: (be_r[b], 0, 0)),
            pl.BlockSpec((1, HIDDEN, MOE_FF),
                         lambda b, be_r, nb_r: (be_r[b], 0, 0)),
            pl.BlockSpec((1, MOE_FF, HIDDEN),
                         lambda b, be_r, nb_r: (be_r[b], 0, 0)),
        ],
        out_specs=pl.BlockSpec((BLK, HIDDEN), lambda b, be_r, nb_r: (b, 0)),
    )
    return pl.pallas_call(
        _k6_body,
        grid_spec=grid_spec,
        out_shape=jax.ShapeDtypeStruct((NROWS, HIDDEN), jnp.float32),
    )(be.reshape(-1), nb.reshape(-1), xs, Weg, Weu, Wed)


def _k7_body(res2_ref, sh_ref, g0_ref, g1_ref, w_ref, out_ref):
    w = w_ref[...]
    out_ref[...] = (res2_ref[...] + sh_ref[...]
                    + w[:, 0:1] * g0_ref[...] + w[:, 1:2] * g1_ref[...])


def _final(res2, shared, g0, g1, w):
    return pl.pallas_call(
        _k7_body,
        grid=(T // BT,),
        in_specs=[
            pl.BlockSpec((BT, HIDDEN), lambda i: (i, 0)),
            pl.BlockSpec((BT, HIDDEN), lambda i: (i, 0)),
            pl.BlockSpec((BT, HIDDEN), lambda i: (i, 0)),
            pl.BlockSpec((BT, HIDDEN), lambda i: (i, 0)),
            pl.BlockSpec((BT, 2), lambda i: (i, 0)),
        ],
        out_specs=pl.BlockSpec((BT, HIDDEN), lambda i: (i, 0)),
        out_shape=jax.ShapeDtypeStruct((T, HIDDEN), jnp.float32),
        compiler_params=pltpu.CompilerParams(
            dimension_semantics=("parallel",)),
    )(res2, shared, g0, g1, w)


@jax.jit
def kernel(positions, hidden_states, Wq, bq, Wk, bk, Wv, bv, Wo, ln1, ln2,
           Wr, Weg, Weu, Wed, Wsg, Wsu, Wsd, Wse):
    q, k, v = _pre_attn(positions, hidden_states, Wq, bq, Wk, bk, Wv, bv, ln1)
    q3 = q.reshape(T, N_HEADS, HEAD_DIM).transpose(1, 0, 2)
    k3 = k.reshape(T, N_KV_HEADS, HEAD_DIM).transpose(1, 0, 2)
    v3 = v.reshape(T, N_KV_HEADS, HEAD_DIM).transpose(1, 0, 2)
    attn3 = _attention(q3, k3, v3)
    attn = attn3.transpose(1, 0, 2).reshape(T, N_HEADS * HEAD_DIM)
    res2, h2, h2b = _post_attn(attn, Wo, hidden_states, ln2)
    w, dest, src, be, nb = _router(h2, Wr)
    xs = _sc_gather(h2, src.reshape(NROWS), NROWS, jnp.float32)
    shared = _shared_expert(h2, Wsg, Wsu, Wsd, Wse)
    ys = _grouped_moe(be, nb, xs, Weg, Weu, Wed)
    g = _sc_gather(ys, dest.T.reshape(2 * T), 2 * T, jnp.float32)
    return _final(res2, shared, g[:T], g[T:], w)


# speedup vs baseline: 1.5833x; 1.1412x over previous
import functools
import jax
import jax.numpy as jnp
from jax.experimental import pallas as pl
from jax.experimental.pallas import tpu as pltpu

HIDDEN = 1024
N_HEADS = 16
N_KV_HEADS = 4
HEAD_DIM = 64
N_EXPERTS = 8
TOP_K = 2
MOE_FF = 1408
SHARED_FF = 2816
EPS = 1e-6
ROPE_BASE = 1000000.0
T = 2048

BT = 256


def _dot(a, b):
    return jax.lax.dot_general(a.astype(jnp.bfloat16), b.astype(jnp.bfloat16),
                               (((1,), (0,)), ((), ())),
                               preferred_element_type=jnp.float32)


def _dot_t(a, b):
    return jax.lax.dot_general(a.astype(jnp.bfloat16), b.astype(jnp.bfloat16),
                               (((1,), (1,)), ((), ())),
                               preferred_element_type=jnp.float32)


def _rms(x, scale):
    var = jnp.mean(jnp.square(x), axis=-1, keepdims=True)
    return x * jax.lax.rsqrt(var + EPS) * scale


def _rope_2d(pos, x, n_heads):
    half = HEAD_DIM // 2
    x3 = x.reshape(x.shape[0], n_heads, HEAD_DIM)
    inv_freq = jnp.exp(
        jnp.arange(0, half, dtype=jnp.int32).astype(jnp.float32)
        * (-jnp.log(ROPE_BASE) / half))
    freqs = pos[:, None] * inv_freq[None, :]
    cos = jnp.cos(freqs)[:, None, :]
    sin = jnp.sin(freqs)[:, None, :]
    x1 = x3[..., :half]
    x2 = x3[..., half:]
    r = jnp.concatenate([x1 * cos - x2 * sin, x2 * cos + x1 * sin], axis=-1)
    return r.reshape(x.shape[0], n_heads * HEAD_DIM)


def _k1_body(pos_ref, h_ref, wq_ref, bq_ref, wk_ref, bk_ref, wv_ref, bv_ref,
             ln1_ref, q_ref, k_ref, v_ref):
    h = _rms(h_ref[...], ln1_ref[...])
    pos = pos_ref[0, 0, :].astype(jnp.float32)
    q = _dot(h, wq_ref[...]) + bq_ref[...]
    k = _dot(h, wk_ref[...]) + bk_ref[...]
    v = _dot(h, wv_ref[...]) + bv_ref[...]
    q_ref[...] = _rope_2d(pos, q, N_HEADS)
    k_ref[...] = _rope_2d(pos, k, N_KV_HEADS)
    v_ref[...] = v


def _pre_attn(positions, hidden_states, Wq, bq, Wk, bk, Wv, bv, ln1):
    pos3 = positions.reshape(T // BT, 1, BT)
    return pl.pallas_call(
        _k1_body,
        grid=(T // BT,),
        in_specs=[
            pl.BlockSpec((1, 1, BT), lambda i: (i, 0, 0)),
            pl.BlockSpec((BT, HIDDEN), lambda i: (i, 0)),
            pl.BlockSpec((HIDDEN, N_HEADS * HEAD_DIM), lambda i: (0, 0)),
            pl.BlockSpec((1, N_HEADS * HEAD_DIM), lambda i: (0, 0)),
            pl.BlockSpec((HIDDEN, N_KV_HEADS * HEAD_DIM), lambda i: (0, 0)),
            pl.BlockSpec((1, N_KV_HEADS * HEAD_DIM), lambda i: (0, 0)),
            pl.BlockSpec((HIDDEN, N_KV_HEADS * HEAD_DIM), lambda i: (0, 0)),
            pl.BlockSpec((1, N_KV_HEADS * HEAD_DIM), lambda i: (0, 0)),
            pl.BlockSpec((1, HIDDEN), lambda i: (0, 0)),
        ],
        out_specs=[
            pl.BlockSpec((BT, N_HEADS * HEAD_DIM), lambda i: (i, 0)),
            pl.BlockSpec((BT, N_KV_HEADS * HEAD_DIM), lambda i: (i, 0)),
            pl.BlockSpec((BT, N_KV_HEADS * HEAD_DIM), lambda i: (i, 0)),
        ],
        out_shape=[
            jax.ShapeDtypeStruct((T, N_HEADS * HEAD_DIM), jnp.float32),
            jax.ShapeDtypeStruct((T, N_KV_HEADS * HEAD_DIM), jnp.float32),
            jax.ShapeDtypeStruct((T, N_KV_HEADS * HEAD_DIM), jnp.float32),
        ],
    )(pos3, hidden_states, Wq, bq.reshape(1, -1), Wk, bk.reshape(1, -1),
      Wv, bv.reshape(1, -1), ln1.reshape(1, -1))


BQ = 1024
CK = 1024
NKJ = T // CK


def _k2_body(q_ref, k_ref, v_ref, o_ref, acc_ref, m_ref, l_ref):
    i = pl.program_id(1)
    j = pl.program_id(2)
    scale = HEAD_DIM ** -0.5

    @pl.when(j == 0)
    def _():
        m_ref[...] = jnp.full(m_ref.shape, -1e30, jnp.float32)
        l_ref[...] = jnp.zeros(l_ref.shape, jnp.float32)
        acc_ref[...] = jnp.zeros(acc_ref.shape, jnp.float32)

    @pl.when(j <= i // (CK // BQ))
    def _():
        q = q_ref[0]
        k = k_ref[0]
        v = v_ref[0]
        s = _dot_t(q, k) * scale
        r = i * BQ + jax.lax.broadcasted_iota(jnp.int32, s.shape, 0)
        c = j * CK + jax.lax.broadcasted_iota(jnp.int32, s.shape, 1)
        s = jnp.where(c <= r, s, jnp.float32(-1e9))
        m_prev = m_ref[...]
        m_cur = jnp.maximum(m_prev, jnp.max(s, axis=-1, keepdims=True))
        alpha = jnp.exp(m_prev - m_cur)
        p = jnp.exp(s - m_cur)
        l_ref[...] = l_ref[...] * alpha + jnp.sum(p, axis=-1, keepdims=True)
        acc_ref[...] = acc_ref[...] * alpha + _dot(p, v)
        m_ref[...] = m_cur

    @pl.when(j == NKJ - 1)
    def _():
        o_ref[0] = acc_ref[...] / l_ref[...]


def _attention(q, k, v):
    rep = N_HEADS // N_KV_HEADS
    rr = CK // BQ

    def kv_idx(h, i, j):
        return (h // rep, jnp.minimum(j, i // rr), 0)

    return pl.pallas_call(
        _k2_body,
        grid=(N_HEADS, T // BQ, NKJ),
        in_specs=[
            pl.BlockSpec((1, BQ, HEAD_DIM), lambda h, i, j: (h, i, 0)),
            pl.BlockSpec((1, CK, HEAD_DIM), kv_idx),
            pl.BlockSpec((1, CK, HEAD_DIM), kv_idx),
        ],
        out_specs=pl.BlockSpec((1, BQ, HEAD_DIM), lambda h, i, j: (h, i, 0)),
        out_shape=jax.ShapeDtypeStruct((N_HEADS, T, HEAD_DIM), jnp.float32),
        scratch_shapes=[
            pltpu.VMEM((BQ, HEAD_DIM), jnp.float32),
            pltpu.VMEM((BQ, 1), jnp.float32),
            pltpu.VMEM((BQ, 1), jnp.float32),
        ],
        compiler_params=pltpu.CompilerParams(
            dimension_semantics=("parallel", "arbitrary", "arbitrary")),
    )(q, k, v)


def _k3_body(attn_ref, wo_ref, res_ref, ln2_ref, res2_ref, h2_ref, h2b_ref):
    hidden = _dot(attn_ref[...], wo_ref[...]) + res_ref[...]
    res2_ref[...] = hidden
    h2 = _rms(hidden, ln2_ref[...])
    h2_ref[...] = h2
    h2b_ref[...] = h2.astype(jnp.bfloat16)


def _post_attn(attn, Wo, residual, ln2):
    return pl.pallas_call(
        _k3_body,
        grid=(T // BT,),
        in_specs=[
            pl.BlockSpec((BT, N_HEADS * HEAD_DIM), lambda i: (i, 0)),
            pl.BlockSpec((N_HEADS * HEAD_DIM, HIDDEN), lambda i: (0, 0)),
            pl.BlockSpec((BT, HIDDEN), lambda i: (i, 0)),
            pl.BlockSpec((1, HIDDEN), lambda i: (0, 0)),
        ],
        out_specs=[
            pl.BlockSpec((BT, HIDDEN), lambda i: (i, 0)),
            pl.BlockSpec((BT, HIDDEN), lambda i: (i, 0)),
            pl.BlockSpec((BT, HIDDEN), lambda i: (i, 0)),
        ],
        out_shape=[
            jax.ShapeDtypeStruct((T, HIDDEN), jnp.float32),
            jax.ShapeDtypeStruct((T, HIDDEN), jnp.float32),
            jax.ShapeDtypeStruct((T, HIDDEN), jnp.bfloat16),
        ],
    )(attn, Wo, residual, ln2.reshape(1, -1))


def _k4_body(h2_ref, wsg_ref, wsu_ref, wsd_ref, wse_ref, out_ref):
    h2 = h2_ref[...]
    g = _dot(h2, wsg_ref[...])
    u = _dot(h2, wsu_ref[...])
    y = _dot(g * jax.lax.logistic(g) * u, wsd_ref[...])
    gate = jax.lax.logistic(_dot(h2, wse_ref[...]))
    out_ref[...] = gate * y


def _shared_expert(h2, Wsg, Wsu, Wsd, Wse):
    return pl.pallas_call(
        _k4_body,
        grid=(T // BT,),
        in_specs=[
            pl.BlockSpec((BT, HIDDEN), lambda i: (i, 0)),
            pl.BlockSpec((HIDDEN, SHARED_FF), lambda i: (0, 0)),
            pl.BlockSpec((HIDDEN, SHARED_FF), lambda i: (0, 0)),
            pl.BlockSpec((SHARED_FF, HIDDEN), lambda i: (0, 0)),
            pl.BlockSpec((HIDDEN, 1), lambda i: (0, 0)),
        ],
        out_specs=pl.BlockSpec((BT, HIDDEN), lambda i: (i, 0)),
        out_shape=jax.ShapeDtypeStruct((T, HIDDEN), jnp.float32),
    )(h2, Wsg, Wsu, Wsd, Wse)


BLK = 256
NBLK = 24
NROWS = NBLK * BLK


def _dotf(a, b):
    return jax.lax.dot_general(a, b, (((1,), (0,)), ((), ())),
                               preferred_element_type=jnp.float32,
                               precision=jax.lax.Precision.HIGHEST)


def _k5_body(h2_ref, wr_ref, w_ref, dest_ref, src_ref, be_ref, nb_ref):
    logits = _dot(h2_ref[...], wr_ref[...])
    m = jnp.max(logits, axis=-1, keepdims=True)
    e = jnp.exp(logits - m)
    probs = e / jnp.sum(e, axis=-1, keepdims=True)
    lane = jax.lax.broadcasted_iota(jnp.int32, probs.shape, 1)
    m1 = jnp.max(probs, axis=-1, keepdims=True)
    i1 = jnp.min(jnp.where(probs == m1, lane, N_EXPERTS), axis=-1,
                 keepdims=True)
    oh1 = (lane == i1).astype(jnp.float32)
    p2 = jnp.where(lane == i1, -1.0, probs)
    m2 = jnp.max(p2, axis=-1, keepdims=True)
    i2 = jnp.min(jnp.where(p2 == m2, lane, N_EXPERTS), axis=-1, keepdims=True)
    oh2 = (lane == i2).astype(jnp.float32)
    denom = m1 + m2
    w_ref[...] = jnp.concatenate([m1 / denom, m2 / denom], axis=1)

    oh = oh1 + oh2
    CH = 256
    tri = (jax.lax.broadcasted_iota(jnp.int32, (CH, CH), 1)
           < jax.lax.broadcasted_iota(jnp.int32, (CH, CH), 0)).astype(
               jnp.float32)
    carry = jnp.zeros((1, N_EXPERTS), jnp.float32)
    ranks = []
    for c in range(T // CH):
        sub = oh[c * CH:(c + 1) * CH, :]
        ranks.append(_dotf(tri, sub) + carry)
        carry = carry + jnp.sum(sub, axis=0, keepdims=True)
    rank = jnp.concatenate(ranks, axis=0)
    counts = carry
    padded = jnp.ceil(counts / BLK) * BLK
    tri8 = (jax.lax.broadcasted_iota(jnp.int32, (N_EXPERTS, N_EXPERTS), 0)
            < jax.lax.broadcasted_iota(jnp.int32, (N_EXPERTS, N_EXPERTS),
                                       1)).astype(jnp.float32)
    off = _dotf(padded, tri8)
    val = off + rank
    d0 = jnp.sum(oh1 * val, axis=1, keepdims=True)
    d1 = jnp.sum(oh2 * val, axis=1, keepdims=True)
    dest_ref[...] = jnp.concatenate([d0, d1], axis=1).astype(jnp.int32)

    d0r = d0.reshape(1, T)
    d1r = d1.reshape(1, T)
    tok = jax.lax.broadcasted_iota(jnp.int32, (T, 1), 0).astype(jnp.float32)
    tokones = jnp.concatenate([tok, jnp.ones_like(tok)], axis=1)

    def _src_block(cb, carry):
        r = (cb * BLK + jax.lax.broadcasted_iota(
            jnp.int32, (BLK, 1), 0)).astype(jnp.float32)
        eq = ((r == d0r) | (r == d1r)).astype(jnp.float32)
        mh = _dotf(eq, tokones)
        matched = mh[:, 0:1]
        hit = mh[:, 1:2]
        filler = r - jnp.floor(r * (1.0 / T)) * T
        src = matched + (1.0 - hit) * filler
        src_ref[pl.ds(cb * BLK, BLK), :] = src.astype(jnp.int32)
        return carry

    jax.lax.fori_loop(0, NBLK, _src_block, 0)

    endb = (off + padded) * (1.0 / BLK)
    b_iota = jax.lax.broadcasted_iota(jnp.int32, (1, NBLK), 1).astype(
        jnp.float32)
    acc = jnp.zeros((1, NBLK), jnp.float32)
    for ei in range(N_EXPERTS):
        acc = acc + (b_iota >= endb[0, ei]).astype(jnp.float32)
    be_ref[...] = jnp.minimum(acc, N_EXPERTS - 1).astype(jnp.int32)
    nb_ref[...] = (jnp.sum(padded, axis=1, keepdims=True)
                   * (1.0 / BLK)).astype(jnp.int32)


def _router(h2, Wr):
    return pl.pallas_call(
        _k5_body,
        grid=(1,),
        in_specs=[
            pl.BlockSpec((T, HIDDEN), lambda i: (0, 0)),
            pl.BlockSpec((HIDDEN, N_EXPERTS), lambda i: (0, 0)),
        ],
        out_specs=[
            pl.BlockSpec((T, 2), lambda i: (0, 0)),
            pl.BlockSpec((T, 2), lambda i: (0, 0)),
            pl.BlockSpec((NROWS, 1), lambda i: (0, 0)),
            pl.BlockSpec((1, NBLK), lambda i: (0, 0)),
            pl.BlockSpec((1, 1), lambda i: (0, 0)),
        ],
        out_shape=[
            jax.ShapeDtypeStruct((T, 2), jnp.float32),
            jax.ShapeDtypeStruct((T, 2), jnp.int32),
            jax.ShapeDtypeStruct((NROWS, 1), jnp.int32),
            jax.ShapeDtypeStruct((1, NBLK), jnp.int32),
            jax.ShapeDtypeStruct((1, 1), jnp.int32),
        ],
    )(h2, Wr)


def _sc_gather(table, idx, n_rows, dtype):
    from jax.experimental.pallas import tpu_sc as plsc
    info = plsc.get_sparse_core_info()
    nw = info.num_cores * info.num_subcores
    b_per_w = n_rows // nw
    ch = 64
    n_ch = b_per_w // ch
    mesh = plsc.VectorSubcoreMesh(core_axis_name="c", subcore_axis_name="s")

    @functools.partial(
        pl.kernel, mesh=mesh,
        out_type=jax.ShapeDtypeStruct((n_rows, HIDDEN), dtype),
        scratch_types=[
            pltpu.VMEM((ch,), jnp.int32),
            pltpu.VMEM((ch, HIDDEN), dtype),
            pltpu.SemaphoreType.DMA,
        ],
    )
    def k(table_hbm, idx_hbm, out_hbm, idx_v, rows_v, sem):
        wid = (jax.lax.axis_index("s") * info.num_cores
               + jax.lax.axis_index("c"))
        base = wid * b_per_w
        for c in range(n_ch):
            off = base + c * ch
            pltpu.sync_copy(idx_hbm.at[pl.ds(off, ch)], idx_v)
            pltpu.async_copy(table_hbm.at[idx_v], rows_v, sem).wait()
            pltpu.sync_copy(rows_v, out_hbm.at[pl.ds(off, ch)])

    return k(table, idx)


def _k6_body(be_ref, nb_ref, xs_ref, weg_ref, weu_ref, wed_ref, ys_ref):
    b = pl.program_id(0)

    @pl.when(b < nb_ref[0])
    def _():
        x = xs_ref[...]
        g = _dot(x, weg_ref[0])
        u = _dot(x, weu_ref[0])
        ys_ref[...] = _dot(g * jax.lax.logistic(g) * u, wed_ref[0])


def _grouped_moe(be, nb, xs, Weg, Weu, Wed):
    grid_spec = pltpu.PrefetchScalarGridSpec(
        num_scalar_prefetch=2,
        grid=(NBLK,),
        in_specs=[
            pl.BlockSpec((BLK, HIDDEN), lambda b, be_r, nb_r: (b, 0)),
            pl.BlockSpec((1, HIDDEN, MOE_FF),
                         lambda b, be_r, nb_r: (be_r[b], 0, 0)),
            pl.BlockSpec((1, HIDDEN, MOE_FF),
                         lambda b, be_r, nb_r: (be_r[b], 0, 0)),
            pl.BlockSpec((1, MOE_FF, HIDDEN),
                         lambda b, be_r, nb_r: (be_r[b], 0, 0)),
        ],
        out_specs=pl.BlockSpec((BLK, HIDDEN), lambda b, be_r, nb_r: (b, 0)),
    )
    return pl.pallas_call(
        _k6_body,
        grid_spec=grid_spec,
        out_shape=jax.ShapeDtypeStruct((NROWS, HIDDEN), jnp.float32),
    )(be.reshape(-1), nb.reshape(-1), xs, Weg, Weu, Wed)


def _k7_body(res2_ref, sh_ref, g0_ref, g1_ref, w_ref, out_ref):
    w = w_ref[...]
    out_ref[...] = (res2_ref[...] + sh_ref[...]
                    + w[:, 0:1] * g0_ref[...] + w[:, 1:2] * g1_ref[...])


def _final(res2, shared, g0, g1, w):
    return pl.pallas_call(
        _k7_body,
        grid=(T // BT,),
        in_specs=[
            pl.BlockSpec((BT, HIDDEN), lambda i: (i, 0)),
            pl.BlockSpec((BT, HIDDEN), lambda i: (i, 0)),
            pl.BlockSpec((BT, HIDDEN), lambda i: (i, 0)),
            pl.BlockSpec((BT, HIDDEN), lambda i: (i, 0)),
            pl.BlockSpec((BT, 2), lambda i: (i, 0)),
        ],
        out_specs=pl.BlockSpec((BT, HIDDEN), lambda i: (i, 0)),
        out_shape=jax.ShapeDtypeStruct((T, HIDDEN), jnp.float32),
        compiler_params=pltpu.CompilerParams(
            dimension_semantics=("parallel",)),
    )(res2, shared, g0, g1, w)


@jax.jit
def kernel(positions, hidden_states, Wq, bq, Wk, bk, Wv, bv, Wo, ln1, ln2,
           Wr, Weg, Weu, Wed, Wsg, Wsu, Wsd, Wse):
    q, k, v = _pre_attn(positions, hidden_states, Wq, bq, Wk, bk, Wv, bv, ln1)
    q3 = q.reshape(T, N_HEADS, HEAD_DIM).transpose(1, 0, 2)
    k3 = k.reshape(T, N_KV_HEADS, HEAD_DIM).transpose(1, 0, 2)
    v3 = v.reshape(T, N_KV_HEADS, HEAD_DIM).transpose(1, 0, 2)
    attn3 = _attention(q3, k3, v3)
    attn = attn3.transpose(1, 0, 2).reshape(T, N_HEADS * HEAD_DIM)
    res2, h2, h2b = _post_attn(attn, Wo, hidden_states, ln2)
    w, dest, src, be, nb = _router(h2, Wr)
    xs = _sc_gather(h2, src.reshape(NROWS), NROWS, jnp.float32)
    shared = _shared_expert(h2, Wsg, Wsu, Wsd, Wse)
    ys = _grouped_moe(be, nb, xs, Weg, Weu, Wed)
    g = _sc_gather(ys, dest.T.reshape(2 * T), 2 * T, jnp.float32)
    return _final(res2, shared, g[:T], g[T:], w)


# speedup vs baseline: 1.5947x; 1.0072x over previous
import functools
import jax
import jax.numpy as jnp
from jax.experimental import pallas as pl
from jax.experimental.pallas import tpu as pltpu

HIDDEN = 1024
N_HEADS = 16
N_KV_HEADS = 4
HEAD_DIM = 64
N_EXPERTS = 8
TOP_K = 2
MOE_FF = 1408
SHARED_FF = 2816
EPS = 1e-6
ROPE_BASE = 1000000.0
T = 2048

BT = 256


def _dot(a, b):
    return jax.lax.dot_general(a.astype(jnp.bfloat16), b.astype(jnp.bfloat16),
                               (((1,), (0,)), ((), ())),
                               preferred_element_type=jnp.float32)


def _dot_t(a, b):
    return jax.lax.dot_general(a.astype(jnp.bfloat16), b.astype(jnp.bfloat16),
                               (((1,), (1,)), ((), ())),
                               preferred_element_type=jnp.float32)


def _rms(x, scale):
    var = jnp.mean(jnp.square(x), axis=-1, keepdims=True)
    return x * jax.lax.rsqrt(var + EPS) * scale


def _rope_2d(pos, x, n_heads):
    half = HEAD_DIM // 2
    x3 = x.reshape(x.shape[0], n_heads, HEAD_DIM)
    inv_freq = jnp.exp(
        jnp.arange(0, half, dtype=jnp.int32).astype(jnp.float32)
        * (-jnp.log(ROPE_BASE) / half))
    freqs = pos[:, None] * inv_freq[None, :]
    cos = jnp.cos(freqs)[:, None, :]
    sin = jnp.sin(freqs)[:, None, :]
    x1 = x3[..., :half]
    x2 = x3[..., half:]
    r = jnp.concatenate([x1 * cos - x2 * sin, x2 * cos + x1 * sin], axis=-1)
    return r.reshape(x.shape[0], n_heads * HEAD_DIM)


def _k1_body(pos_ref, h_ref, wq_ref, bq_ref, wk_ref, bk_ref, wv_ref, bv_ref,
             ln1_ref, q_ref, k_ref, v_ref):
    h = _rms(h_ref[...], ln1_ref[...])
    pos = pos_ref[0, 0, :].astype(jnp.float32)
    q = _dot(h, wq_ref[...]) + bq_ref[...]
    k = _dot(h, wk_ref[...]) + bk_ref[...]
    v = _dot(h, wv_ref[...]) + bv_ref[...]
    q_ref[...] = _rope_2d(pos, q, N_HEADS)
    k_ref[...] = _rope_2d(pos, k, N_KV_HEADS)
    v_ref[...] = v


def _pre_attn(positions, hidden_states, Wq, bq, Wk, bk, Wv, bv, ln1):
    pos3 = positions.reshape(T // BT, 1, BT)
    return pl.pallas_call(
        _k1_body,
        grid=(T // BT,),
        in_specs=[
            pl.BlockSpec((1, 1, BT), lambda i: (i, 0, 0)),
            pl.BlockSpec((BT, HIDDEN), lambda i: (i, 0)),
            pl.BlockSpec((HIDDEN, N_HEADS * HEAD_DIM), lambda i: (0, 0)),
            pl.BlockSpec((1, N_HEADS * HEAD_DIM), lambda i: (0, 0)),
            pl.BlockSpec((HIDDEN, N_KV_HEADS * HEAD_DIM), lambda i: (0, 0)),
            pl.BlockSpec((1, N_KV_HEADS * HEAD_DIM), lambda i: (0, 0)),
            pl.BlockSpec((HIDDEN, N_KV_HEADS * HEAD_DIM), lambda i: (0, 0)),
            pl.BlockSpec((1, N_KV_HEADS * HEAD_DIM), lambda i: (0, 0)),
            pl.BlockSpec((1, HIDDEN), lambda i: (0, 0)),
        ],
        out_specs=[
            pl.BlockSpec((BT, N_HEADS * HEAD_DIM), lambda i: (i, 0)),
            pl.BlockSpec((BT, N_KV_HEADS * HEAD_DIM), lambda i: (i, 0)),
            pl.BlockSpec((BT, N_KV_HEADS * HEAD_DIM), lambda i: (i, 0)),
        ],
        out_shape=[
            jax.ShapeDtypeStruct((T, N_HEADS * HEAD_DIM), jnp.float32),
            jax.ShapeDtypeStruct((T, N_KV_HEADS * HEAD_DIM), jnp.float32),
            jax.ShapeDtypeStruct((T, N_KV_HEADS * HEAD_DIM), jnp.float32),
        ],
    )(pos3, hidden_states, Wq, bq.reshape(1, -1), Wk, bk.reshape(1, -1),
      Wv, bv.reshape(1, -1), ln1.reshape(1, -1))


BQ = 1024
CK = 2048
NKJ = T // CK


def _k2_body(q_ref, k_ref, v_ref, o_ref, acc_ref, m_ref, l_ref):
    i = pl.program_id(1)
    j = pl.program_id(2)
    scale = HEAD_DIM ** -0.5

    @pl.when(j == 0)
    def _():
        m_ref[...] = jnp.full(m_ref.shape, -1e30, jnp.float32)
        l_ref[...] = jnp.zeros(l_ref.shape, jnp.float32)
        acc_ref[...] = jnp.zeros(acc_ref.shape, jnp.float32)

    @pl.when(j <= i // (CK // BQ))
    def _():
        q = q_ref[0]
        k = k_ref[0]
        v = v_ref[0]
        s = _dot_t(q, k) * scale
        r = i * BQ + jax.lax.broadcasted_iota(jnp.int32, s.shape, 0)
        c = j * CK + jax.lax.broadcasted_iota(jnp.int32, s.shape, 1)
        s = jnp.where(c <= r, s, jnp.float32(-1e9))
        m_prev = m_ref[...]
        m_cur = jnp.maximum(m_prev, jnp.max(s, axis=-1, keepdims=True))
        alpha = jnp.exp(m_prev - m_cur)
        p = jnp.exp(s - m_cur)
        l_ref[...] = l_ref[...] * alpha + jnp.sum(p, axis=-1, keepdims=True)
        acc_ref[...] = acc_ref[...] * alpha + _dot(p, v)
        m_ref[...] = m_cur

    @pl.when(j == NKJ - 1)
    def _():
        o_ref[0] = acc_ref[...] / l_ref[...]


def _attention(q, k, v):
    rep = N_HEADS // N_KV_HEADS
    rr = CK // BQ

    def kv_idx(h, i, j):
        return (h // rep, jnp.minimum(j, i // rr), 0)

    return pl.pallas_call(
        _k2_body,
        grid=(N_HEADS, T // BQ, NKJ),
        in_specs=[
            pl.BlockSpec((1, BQ, HEAD_DIM), lambda h, i, j: (h, i, 0)),
            pl.BlockSpec((1, CK, HEAD_DIM), kv_idx),
            pl.BlockSpec((1, CK, HEAD_DIM), kv_idx),
        ],
        out_specs=pl.BlockSpec((1, BQ, HEAD_DIM), lambda h, i, j: (h, i, 0)),
        out_shape=jax.ShapeDtypeStruct((N_HEADS, T, HEAD_DIM), jnp.float32),
        scratch_shapes=[
            pltpu.VMEM((BQ, HEAD_DIM), jnp.float32),
            pltpu.VMEM((BQ, 1), jnp.float32),
            pltpu.VMEM((BQ, 1), jnp.float32),
        ],
        compiler_params=pltpu.CompilerParams(
            dimension_semantics=("parallel", "arbitrary", "arbitrary")),
    )(q, k, v)


def _k3_body(attn_ref, wo_ref, res_ref, ln2_ref, res2_ref, h2_ref, h2b_ref):
    hidden = _dot(attn_ref[...], wo_ref[...]) + res_ref[...]
    res2_ref[...] = hidden
    h2 = _rms(hidden, ln2_ref[...])
    h2_ref[...] = h2
    h2b_ref[...] = h2.astype(jnp.bfloat16)


def _post_attn(attn, Wo, residual, ln2):
    return pl.pallas_call(
        _k3_body,
        grid=(T // BT,),
        in_specs=[
            pl.BlockSpec((BT, N_HEADS * HEAD_DIM), lambda i: (i, 0)),
            pl.BlockSpec((N_HEADS * HEAD_DIM, HIDDEN), lambda i: (0, 0)),
            pl.BlockSpec((BT, HIDDEN), lambda i: (i, 0)),
            pl.BlockSpec((1, HIDDEN), lambda i: (0, 0)),
        ],
        out_specs=[
            pl.BlockSpec((BT, HIDDEN), lambda i: (i, 0)),
            pl.BlockSpec((BT, HIDDEN), lambda i: (i, 0)),
            pl.BlockSpec((BT, HIDDEN), lambda i: (i, 0)),
        ],
        out_shape=[
            jax.ShapeDtypeStruct((T, HIDDEN), jnp.float32),
            jax.ShapeDtypeStruct((T, HIDDEN), jnp.float32),
            jax.ShapeDtypeStruct((T, HIDDEN), jnp.bfloat16),
        ],
    )(attn, Wo, residual, ln2.reshape(1, -1))


def _k4_body(h2_ref, wsg_ref, wsu_ref, wsd_ref, wse_ref, out_ref):
    h2 = h2_ref[...]
    g = _dot(h2, wsg_ref[...])
    u = _dot(h2, wsu_ref[...])
    y = _dot(g * jax.lax.logistic(g) * u, wsd_ref[...])
    gate = jax.lax.logistic(_dot(h2, wse_ref[...]))
    out_ref[...] = gate * y


def _shared_expert(h2, Wsg, Wsu, Wsd, Wse):
    return pl.pallas_call(
        _k4_body,
        grid=(T // BT,),
        in_specs=[
            pl.BlockSpec((BT, HIDDEN), lambda i: (i, 0)),
            pl.BlockSpec((HIDDEN, SHARED_FF), lambda i: (0, 0)),
            pl.BlockSpec((HIDDEN, SHARED_FF), lambda i: (0, 0)),
            pl.BlockSpec((SHARED_FF, HIDDEN), lambda i: (0, 0)),
            pl.BlockSpec((HIDDEN, 1), lambda i: (0, 0)),
        ],
        out_specs=pl.BlockSpec((BT, HIDDEN), lambda i: (i, 0)),
        out_shape=jax.ShapeDtypeStruct((T, HIDDEN), jnp.float32),
    )(h2, Wsg, Wsu, Wsd, Wse)


BLK = 256
NBLK = 24
NROWS = NBLK * BLK


def _dotf(a, b):
    return jax.lax.dot_general(a, b, (((1,), (0,)), ((), ())),
                               preferred_element_type=jnp.float32,
                               precision=jax.lax.Precision.HIGHEST)


def _k5_body(h2_ref, wr_ref, w_ref, dest_ref, src_ref, be_ref, nb_ref):
    logits = _dot(h2_ref[...], wr_ref[...])
    m = jnp.max(logits, axis=-1, keepdims=True)
    e = jnp.exp(logits - m)
    probs = e / jnp.sum(e, axis=-1, keepdims=True)
    lane = jax.lax.broadcasted_iota(jnp.int32, probs.shape, 1)
    m1 = jnp.max(probs, axis=-1, keepdims=True)
    i1 = jnp.min(jnp.where(probs == m1, lane, N_EXPERTS), axis=-1,
                 keepdims=True)
    oh1 = (lane == i1).astype(jnp.float32)
    p2 = jnp.where(lane == i1, -1.0, probs)
    m2 = jnp.max(p2, axis=-1, keepdims=True)
    i2 = jnp.min(jnp.where(p2 == m2, lane, N_EXPERTS), axis=-1, keepdims=True)
    oh2 = (lane == i2).astype(jnp.float32)
    denom = m1 + m2
    w_ref[...] = jnp.concatenate([m1 / denom, m2 / denom], axis=1)

    oh = oh1 + oh2
    CH = 256
    tri = (jax.lax.broadcasted_iota(jnp.int32, (CH, CH), 1)
           < jax.lax.broadcasted_iota(jnp.int32, (CH, CH), 0)).astype(
               jnp.float32)
    carry = jnp.zeros((1, N_EXPERTS), jnp.float32)
    ranks = []
    for c in range(T // CH):
        sub = oh[c * CH:(c + 1) * CH, :]
        ranks.append(_dotf(tri, sub) + carry)
        carry = carry + jnp.sum(sub, axis=0, keepdims=True)
    rank = jnp.concatenate(ranks, axis=0)
    counts = carry
    padded = jnp.ceil(counts / BLK) * BLK
    tri8 = (jax.lax.broadcasted_iota(jnp.int32, (N_EXPERTS, N_EXPERTS), 0)
            < jax.lax.broadcasted_iota(jnp.int32, (N_EXPERTS, N_EXPERTS),
                                       1)).astype(jnp.float32)
    off = _dotf(padded, tri8)
    val = off + rank
    d0 = jnp.sum(oh1 * val, axis=1, keepdims=True)
    d1 = jnp.sum(oh2 * val, axis=1, keepdims=True)
    dest_ref[...] = jnp.concatenate([d0, d1], axis=1).astype(jnp.int32)

    d0r = d0.reshape(1, T)
    d1r = d1.reshape(1, T)
    tok = jax.lax.broadcasted_iota(jnp.int32, (T, 1), 0).astype(jnp.float32)
    tokones = jnp.concatenate([tok, jnp.ones_like(tok)], axis=1)

    def _src_block(cb, carry):
        r = (cb * BLK + jax.lax.broadcasted_iota(
            jnp.int32, (BLK, 1), 0)).astype(jnp.float32)
        eq = ((r == d0r) | (r == d1r)).astype(jnp.float32)
        mh = _dotf(eq, tokones)
        matched = mh[:, 0:1]
        hit = mh[:, 1:2]
        filler = r - jnp.floor(r * (1.0 / T)) * T
        src = matched + (1.0 - hit) * filler
        src_ref[pl.ds(cb * BLK, BLK), :] = src.astype(jnp.int32)
        return carry

    jax.lax.fori_loop(0, NBLK, _src_block, 0)

    endb = (off + padded) * (1.0 / BLK)
    b_iota = jax.lax.broadcasted_iota(jnp.int32, (1, NBLK), 1).astype(
        jnp.float32)
    acc = jnp.zeros((1, NBLK), jnp.float32)
    for ei in range(N_EXPERTS):
        acc = acc + (b_iota >= endb[0, ei]).astype(jnp.float32)
    be_ref[...] = jnp.minimum(acc, N_EXPERTS - 1).astype(jnp.int32)
    nb_ref[...] = (jnp.sum(padded, axis=1, keepdims=True)
                   * (1.0 / BLK)).astype(jnp.int32)


def _router(h2, Wr):
    return pl.pallas_call(
        _k5_body,
        grid=(1,),
        in_specs=[
            pl.BlockSpec((T, HIDDEN), lambda i: (0, 0)),
            pl.BlockSpec((HIDDEN, N_EXPERTS), lambda i: (0, 0)),
        ],
        out_specs=[
            pl.BlockSpec((T, 2), lambda i: (0, 0)),
            pl.BlockSpec((T, 2), lambda i: (0, 0)),
            pl.BlockSpec((NROWS, 1), lambda i: (0, 0)),
            pl.BlockSpec((1, NBLK), lambda i: (0, 0)),
            pl.BlockSpec((1, 1), lambda i: (0, 0)),
        ],
        out_shape=[
            jax.ShapeDtypeStruct((T, 2), jnp.float32),
            jax.ShapeDtypeStruct((T, 2), jnp.int32),
            jax.ShapeDtypeStruct((NROWS, 1), jnp.int32),
            jax.ShapeDtypeStruct((1, NBLK), jnp.int32),
            jax.ShapeDtypeStruct((1, 1), jnp.int32),
        ],
    )(h2, Wr)


def _sc_gather(table, idx, n_rows, dtype):
    from jax.experimental.pallas import tpu_sc as plsc
    info = plsc.get_sparse_core_info()
    nw = info.num_cores * info.num_subcores
    b_per_w = n_rows // nw
    ch = 64
    n_ch = b_per_w // ch
    mesh = plsc.VectorSubcoreMesh(core_axis_name="c", subcore_axis_name="s")

    @functools.partial(
        pl.kernel, mesh=mesh,
        out_type=jax.ShapeDtypeStruct((n_rows, HIDDEN), dtype),
        scratch_types=[
            pltpu.VMEM((ch,), jnp.int32),
            pltpu.VMEM((ch, HIDDEN), dtype),
            pltpu.SemaphoreType.DMA,
        ],
    )
    def k(table_hbm, idx_hbm, out_hbm, idx_v, rows_v, sem):
        wid = (jax.lax.axis_index("s") * info.num_cores
               + jax.lax.axis_index("c"))
        base = wid * b_per_w
        for c in range(n_ch):
            off = base + c * ch
            pltpu.sync_copy(idx_hbm.at[pl.ds(off, ch)], idx_v)
            pltpu.async_copy(table_hbm.at[idx_v], rows_v, sem).wait()
            pltpu.sync_copy(rows_v, out_hbm.at[pl.ds(off, ch)])

    return k(table, idx)


def _k6_body(be_ref, nb_ref, xs_ref, weg_ref, weu_ref, wed_ref, ys_ref):
    b = pl.program_id(0)

    @pl.when(b < nb_ref[0])
    def _():
        x = xs_ref[...]
        g = _dot(x, weg_ref[0])
        u = _dot(x, weu_ref[0])
        ys_ref[...] = _dot(g * jax.lax.logistic(g) * u, wed_ref[0])


def _grouped_moe(be, nb, xs, Weg, Weu, Wed):
    grid_spec = pltpu.PrefetchScalarGridSpec(
        num_scalar_prefetch=2,
        grid=(NBLK,),
        in_specs=[
            pl.BlockSpec((BLK, HIDDEN), lambda b, be_r, nb_r: (b, 0)),
            pl.BlockSpec((1, HIDDEN, MOE_FF),
                         lambda b, be_r, nb_r: (be_r[b], 0, 0)),
            pl.BlockSpec((1, HIDDEN, MOE_FF),
                         lambda b, be_r, nb_r: (be_r[b], 0, 0)),
            pl.BlockSpec((1, MOE_FF, HIDDEN),
                         lambda b, be_r, nb_r: (be_r[b], 0, 0)),
        ],
        out_specs=pl.BlockSpec((BLK, HIDDEN), lambda b, be_r, nb_r: (b, 0)),
    )
    return pl.pallas_call(
        _k6_body,
        grid_spec=grid_spec,
        out_shape=jax.ShapeDtypeStruct((NROWS, HIDDEN), jnp.float32),
    )(be.reshape(-1), nb.reshape(-1), xs, Weg, Weu, Wed)


def _k7_body(res2_ref, sh_ref, g0_ref, g1_ref, w_ref, out_ref):
    w = w_ref[...]
    out_ref[...] = (res2_ref[...] + sh_ref[...]
                    + w[:, 0:1] * g0_ref[...] + w[:, 1:2] * g1_ref[...])


def _final(res2, shared, g0, g1, w):
    return pl.pallas_call(
        _k7_body,
        grid=(T // BT,),
        in_specs=[
            pl.BlockSpec((BT, HIDDEN), lambda i: (i, 0)),
            pl.BlockSpec((BT, HIDDEN), lambda i: (i, 0)),
            pl.BlockSpec((BT, HIDDEN), lambda i: (i, 0)),
            pl.BlockSpec((BT, HIDDEN), lambda i: (i, 0)),
            pl.BlockSpec((BT, 2), lambda i: (i, 0)),
        ],
        out_specs=pl.BlockSpec((BT, HIDDEN), lambda i: (i, 0)),
        out_shape=jax.ShapeDtypeStruct((T, HIDDEN), jnp.float32),
        compiler_params=pltpu.CompilerParams(
            dimension_semantics=("parallel",)),
    )(res2, shared, g0, g1, w)


@jax.jit
def kernel(positions, hidden_states, Wq, bq, Wk, bk, Wv, bv, Wo, ln1, ln2,
           Wr, Weg, Weu, Wed, Wsg, Wsu, Wsd, Wse):
    q, k, v = _pre_attn(positions, hidden_states, Wq, bq, Wk, bk, Wv, bv, ln1)
    q3 = q.reshape(T, N_HEADS, HEAD_DIM).transpose(1, 0, 2)
    k3 = k.reshape(T, N_KV_HEADS, HEAD_DIM).transpose(1, 0, 2)
    v3 = v.reshape(T, N_KV_HEADS, HEAD_DIM).transpose(1, 0, 2)
    attn3 = _attention(q3, k3, v3)
    attn = attn3.transpose(1, 0, 2).reshape(T, N_HEADS * HEAD_DIM)
    res2, h2, h2b = _post_attn(attn, Wo, hidden_states, ln2)
    w, dest, src, be, nb = _router(h2, Wr)
    xs = _sc_gather(h2, src.reshape(NROWS), NROWS, jnp.float32)
    shared = _shared_expert(h2, Wsg, Wsu, Wsd, Wse)
    ys = _grouped_moe(be, nb, xs, Weg, Weu, Wed)
    g = _sc_gather(ys, dest.T.reshape(2 * T), 2 * T, jnp.float32)
    return _final(res2, shared, g[:T], g[T:], w)


# speedup vs baseline: 1.6008x; 1.0038x over previous
import functools
import jax
import jax.numpy as jnp
from jax.experimental import pallas as pl
from jax.experimental.pallas import tpu as pltpu

HIDDEN = 1024
N_HEADS = 16
N_KV_HEADS = 4
HEAD_DIM = 64
N_EXPERTS = 8
TOP_K = 2
MOE_FF = 1408
SHARED_FF = 2816
EPS = 1e-6
ROPE_BASE = 1000000.0
T = 2048

BT = 512


def _dot(a, b):
    return jax.lax.dot_general(a.astype(jnp.bfloat16), b.astype(jnp.bfloat16),
                               (((1,), (0,)), ((), ())),
                               preferred_element_type=jnp.float32)


def _dot_t(a, b):
    return jax.lax.dot_general(a.astype(jnp.bfloat16), b.astype(jnp.bfloat16),
                               (((1,), (1,)), ((), ())),
                               preferred_element_type=jnp.float32)


def _rms(x, scale):
    var = jnp.mean(jnp.square(x), axis=-1, keepdims=True)
    return x * jax.lax.rsqrt(var + EPS) * scale


def _rope_2d(pos, x, n_heads):
    half = HEAD_DIM // 2
    x3 = x.reshape(x.shape[0], n_heads, HEAD_DIM)
    inv_freq = jnp.exp(
        jnp.arange(0, half, dtype=jnp.int32).astype(jnp.float32)
        * (-jnp.log(ROPE_BASE) / half))
    freqs = pos[:, None] * inv_freq[None, :]
    cos = jnp.cos(freqs)[:, None, :]
    sin = jnp.sin(freqs)[:, None, :]
    x1 = x3[..., :half]
    x2 = x3[..., half:]
    r = jnp.concatenate([x1 * cos - x2 * sin, x2 * cos + x1 * sin], axis=-1)
    return r.reshape(x.shape[0], n_heads * HEAD_DIM)


def _k1_body(pos_ref, h_ref, wq_ref, bq_ref, wk_ref, bk_ref, wv_ref, bv_ref,
             ln1_ref, q_ref, k_ref, v_ref):
    h = _rms(h_ref[...], ln1_ref[...])
    pos = pos_ref[0, 0, :].astype(jnp.float32)
    q = _dot(h, wq_ref[...]) + bq_ref[...]
    k = _dot(h, wk_ref[...]) + bk_ref[...]
    v = _dot(h, wv_ref[...]) + bv_ref[...]
    q_ref[...] = _rope_2d(pos, q, N_HEADS)
    k_ref[...] = _rope_2d(pos, k, N_KV_HEADS)
    v_ref[...] = v


def _pre_attn(positions, hidden_states, Wq, bq, Wk, bk, Wv, bv, ln1):
    pos3 = positions.reshape(T // BT, 1, BT)
    return pl.pallas_call(
        _k1_body,
        grid=(T // BT,),
        in_specs=[
            pl.BlockSpec((1, 1, BT), lambda i: (i, 0, 0)),
            pl.BlockSpec((BT, HIDDEN), lambda i: (i, 0)),
            pl.BlockSpec((HIDDEN, N_HEADS * HEAD_DIM), lambda i: (0, 0)),
            pl.BlockSpec((1, N_HEADS * HEAD_DIM), lambda i: (0, 0)),
            pl.BlockSpec((HIDDEN, N_KV_HEADS * HEAD_DIM), lambda i: (0, 0)),
            pl.BlockSpec((1, N_KV_HEADS * HEAD_DIM), lambda i: (0, 0)),
            pl.BlockSpec((HIDDEN, N_KV_HEADS * HEAD_DIM), lambda i: (0, 0)),
            pl.BlockSpec((1, N_KV_HEADS * HEAD_DIM), lambda i: (0, 0)),
            pl.BlockSpec((1, HIDDEN), lambda i: (0, 0)),
        ],
        out_specs=[
            pl.BlockSpec((BT, N_HEADS * HEAD_DIM), lambda i: (i, 0)),
            pl.BlockSpec((BT, N_KV_HEADS * HEAD_DIM), lambda i: (i, 0)),
            pl.BlockSpec((BT, N_KV_HEADS * HEAD_DIM), lambda i: (i, 0)),
        ],
        out_shape=[
            jax.ShapeDtypeStruct((T, N_HEADS * HEAD_DIM), jnp.float32),
            jax.ShapeDtypeStruct((T, N_KV_HEADS * HEAD_DIM), jnp.float32),
            jax.ShapeDtypeStruct((T, N_KV_HEADS * HEAD_DIM), jnp.float32),
        ],
    )(pos3, hidden_states, Wq, bq.reshape(1, -1), Wk, bk.reshape(1, -1),
      Wv, bv.reshape(1, -1), ln1.reshape(1, -1))


BQ = 1024
CK = 2048
NKJ = T // CK


def _k2_body(q_ref, k_ref, v_ref, o_ref, acc_ref, m_ref, l_ref):
    i = pl.program_id(1)
    j = pl.program_id(2)
    scale = HEAD_DIM ** -0.5

    @pl.when(j == 0)
    def _():
        m_ref[...] = jnp.full(m_ref.shape, -1e30, jnp.float32)
        l_ref[...] = jnp.zeros(l_ref.shape, jnp.float32)
        acc_ref[...] = jnp.zeros(acc_ref.shape, jnp.float32)

    @pl.when(j <= i // (CK // BQ))
    def _():
        q = q_ref[0]
        k = k_ref[0]
        v = v_ref[0]
        s = _dot_t(q, k) * scale
        r = i * BQ + jax.lax.broadcasted_iota(jnp.int32, s.shape, 0)
        c = j * CK + jax.lax.broadcasted_iota(jnp.int32, s.shape, 1)
        s = jnp.where(c <= r, s, jnp.float32(-1e9))
        m_prev = m_ref[...]
        m_cur = jnp.maximum(m_prev, jnp.max(s, axis=-1, keepdims=True))
        alpha = jnp.exp(m_prev - m_cur)
        p = jnp.exp(s - m_cur)
        l_ref[...] = l_ref[...] * alpha + jnp.sum(p, axis=-1, keepdims=True)
        acc_ref[...] = acc_ref[...] * alpha + _dot(p, v)
        m_ref[...] = m_cur

    @pl.when(j == NKJ - 1)
    def _():
        o_ref[0] = acc_ref[...] / l_ref[...]


def _attention(q, k, v):
    rep = N_HEADS // N_KV_HEADS
    rr = CK // BQ

    def kv_idx(h, i, j):
        return (h // rep, jnp.minimum(j, i // rr), 0)

    return pl.pallas_call(
        _k2_body,
        grid=(N_HEADS, T // BQ, NKJ),
        in_specs=[
            pl.BlockSpec((1, BQ, HEAD_DIM), lambda h, i, j: (h, i, 0)),
            pl.BlockSpec((1, CK, HEAD_DIM), kv_idx),
            pl.BlockSpec((1, CK, HEAD_DIM), kv_idx),
        ],
        out_specs=pl.BlockSpec((1, BQ, HEAD_DIM), lambda h, i, j: (h, i, 0)),
        out_shape=jax.ShapeDtypeStruct((N_HEADS, T, HEAD_DIM), jnp.float32),
        scratch_shapes=[
            pltpu.VMEM((BQ, HEAD_DIM), jnp.float32),
            pltpu.VMEM((BQ, 1), jnp.float32),
            pltpu.VMEM((BQ, 1), jnp.float32),
        ],
        compiler_params=pltpu.CompilerParams(
            dimension_semantics=("parallel", "arbitrary", "arbitrary")),
    )(q, k, v)


def _k3_body(attn_ref, wo_ref, res_ref, ln2_ref, res2_ref, h2_ref, h2b_ref):
    hidden = _dot(attn_ref[...], wo_ref[...]) + res_ref[...]
    res2_ref[...] = hidden
    h2 = _rms(hidden, ln2_ref[...])
    h2_ref[...] = h2
    h2b_ref[...] = h2.astype(jnp.bfloat16)


def _post_attn(attn, Wo, residual, ln2):
    return pl.pallas_call(
        _k3_body,
        grid=(T // BT,),
        in_specs=[
            pl.BlockSpec((BT, N_HEADS * HEAD_DIM), lambda i: (i, 0)),
            pl.BlockSpec((N_HEADS * HEAD_DIM, HIDDEN), lambda i: (0, 0)),
            pl.BlockSpec((BT, HIDDEN), lambda i: (i, 0)),
            pl.BlockSpec((1, HIDDEN), lambda i: (0, 0)),
        ],
        out_specs=[
            pl.BlockSpec((BT, HIDDEN), lambda i: (i, 0)),
            pl.BlockSpec((BT, HIDDEN), lambda i: (i, 0)),
            pl.BlockSpec((BT, HIDDEN), lambda i: (i, 0)),
        ],
        out_shape=[
            jax.ShapeDtypeStruct((T, HIDDEN), jnp.float32),
            jax.ShapeDtypeStruct((T, HIDDEN), jnp.float32),
            jax.ShapeDtypeStruct((T, HIDDEN), jnp.bfloat16),
        ],
    )(attn, Wo, residual, ln2.reshape(1, -1))


def _k4_body(h2_ref, wsg_ref, wsu_ref, wsd_ref, wse_ref, out_ref):
    h2 = h2_ref[...]
    g = _dot(h2, wsg_ref[...])
    u = _dot(h2, wsu_ref[...])
    y = _dot(g * jax.lax.logistic(g) * u, wsd_ref[...])
    gate = jax.lax.logistic(_dot(h2, wse_ref[...]))
    out_ref[...] = gate * y


def _shared_expert(h2, Wsg, Wsu, Wsd, Wse):
    return pl.pallas_call(
        _k4_body,
        grid=(T // BT,),
        in_specs=[
            pl.BlockSpec((BT, HIDDEN), lambda i: (i, 0)),
            pl.BlockSpec((HIDDEN, SHARED_FF), lambda i: (0, 0)),
            pl.BlockSpec((HIDDEN, SHARED_FF), lambda i: (0, 0)),
            pl.BlockSpec((SHARED_FF, HIDDEN), lambda i: (0, 0)),
            pl.BlockSpec((HIDDEN, 1), lambda i: (0, 0)),
        ],
        out_specs=pl.BlockSpec((BT, HIDDEN), lambda i: (i, 0)),
        out_shape=jax.ShapeDtypeStruct((T, HIDDEN), jnp.float32),
    )(h2, Wsg, Wsu, Wsd, Wse)


BLK = 256
NBLK = 24
NROWS = NBLK * BLK


def _dotf(a, b):
    return jax.lax.dot_general(a, b, (((1,), (0,)), ((), ())),
                               preferred_element_type=jnp.float32,
                               precision=jax.lax.Precision.HIGHEST)


def _k5_body(h2_ref, wr_ref, w_ref, dest_ref, src_ref, be_ref, nb_ref):
    logits = _dot(h2_ref[...], wr_ref[...])
    m = jnp.max(logits, axis=-1, keepdims=True)
    e = jnp.exp(logits - m)
    probs = e / jnp.sum(e, axis=-1, keepdims=True)
    lane = jax.lax.broadcasted_iota(jnp.int32, probs.shape, 1)
    m1 = jnp.max(probs, axis=-1, keepdims=True)
    i1 = jnp.min(jnp.where(probs == m1, lane, N_EXPERTS), axis=-1,
                 keepdims=True)
    oh1 = (lane == i1).astype(jnp.float32)
    p2 = jnp.where(lane == i1, -1.0, probs)
    m2 = jnp.max(p2, axis=-1, keepdims=True)
    i2 = jnp.min(jnp.where(p2 == m2, lane, N_EXPERTS), axis=-1, keepdims=True)
    oh2 = (lane == i2).astype(jnp.float32)
    denom = m1 + m2
    w_ref[...] = jnp.concatenate([m1 / denom, m2 / denom], axis=1)

    oh = oh1 + oh2
    CH = 256
    tri = (jax.lax.broadcasted_iota(jnp.int32, (CH, CH), 1)
           < jax.lax.broadcasted_iota(jnp.int32, (CH, CH), 0)).astype(
               jnp.float32)
    carry = jnp.zeros((1, N_EXPERTS), jnp.float32)
    ranks = []
    for c in range(T // CH):
        sub = oh[c * CH:(c + 1) * CH, :]
        ranks.append(_dotf(tri, sub) + carry)
        carry = carry + jnp.sum(sub, axis=0, keepdims=True)
    rank = jnp.concatenate(ranks, axis=0)
    counts = carry
    padded = jnp.ceil(counts / BLK) * BLK
    tri8 = (jax.lax.broadcasted_iota(jnp.int32, (N_EXPERTS, N_EXPERTS), 0)
            < jax.lax.broadcasted_iota(jnp.int32, (N_EXPERTS, N_EXPERTS),
                                       1)).astype(jnp.float32)
    off = _dotf(padded, tri8)
    val = off + rank
    d0 = jnp.sum(oh1 * val, axis=1, keepdims=True)
    d1 = jnp.sum(oh2 * val, axis=1, keepdims=True)
    dest_ref[...] = jnp.concatenate([d0, d1], axis=1).astype(jnp.int32)

    d0r = d0.reshape(1, T)
    d1r = d1.reshape(1, T)
    tok = jax.lax.broadcasted_iota(jnp.int32, (T, 1), 0).astype(jnp.float32)
    tokones = jnp.concatenate([tok, jnp.ones_like(tok)], axis=1)

    def _src_block(cb, carry):
        r = (cb * BLK + jax.lax.broadcasted_iota(
            jnp.int32, (BLK, 1), 0)).astype(jnp.float32)
        eq = ((r == d0r) | (r == d1r)).astype(jnp.float32)
        mh = _dotf(eq, tokones)
        matched = mh[:, 0:1]
        hit = mh[:, 1:2]
        filler = r - jnp.floor(r * (1.0 / T)) * T
        src = matched + (1.0 - hit) * filler
        src_ref[pl.ds(cb * BLK, BLK), :] = src.astype(jnp.int32)
        return carry

    jax.lax.fori_loop(0, NBLK, _src_block, 0)

    endb = (off + padded) * (1.0 / BLK)
    b_iota = jax.lax.broadcasted_iota(jnp.int32, (1, NBLK), 1).astype(
        jnp.float32)
    acc = jnp.zeros((1, NBLK), jnp.float32)
    for ei in range(N_EXPERTS):
        acc = acc + (b_iota >= endb[0, ei]).astype(jnp.float32)
    be_ref[...] = jnp.minimum(acc, N_EXPERTS - 1).astype(jnp.int32)
    nb_ref[...] = (jnp.sum(padded, axis=1, keepdims=True)
                   * (1.0 / BLK)).astype(jnp.int32)


def _router(h2, Wr):
    return pl.pallas_call(
        _k5_body,
        grid=(1,),
        in_specs=[
            pl.BlockSpec((T, HIDDEN), lambda i: (0, 0)),
            pl.BlockSpec((HIDDEN, N_EXPERTS), lambda i: (0, 0)),
        ],
        out_specs=[
            pl.BlockSpec((T, 2), lambda i: (0, 0)),
            pl.BlockSpec((T, 2), lambda i: (0, 0)),
            pl.BlockSpec((NROWS, 1), lambda i: (0, 0)),
            pl.BlockSpec((1, NBLK), lambda i: (0, 0)),
            pl.BlockSpec((1, 1), lambda i: (0, 0)),
        ],
        out_shape=[
            jax.ShapeDtypeStruct((T, 2), jnp.float32),
            jax.ShapeDtypeStruct((T, 2), jnp.int32),
            jax.ShapeDtypeStruct((NROWS, 1), jnp.int32),
            jax.ShapeDtypeStruct((1, NBLK), jnp.int32),
            jax.ShapeDtypeStruct((1, 1), jnp.int32),
        ],
    )(h2, Wr)


def _sc_gather(table, idx, n_rows, dtype):
    from jax.experimental.pallas import tpu_sc as plsc
    info = plsc.get_sparse_core_info()
    nw = info.num_cores * info.num_subcores
    b_per_w = n_rows // nw
    ch = 64
    n_ch = b_per_w // ch
    mesh = plsc.VectorSubcoreMesh(core_axis_name="c", subcore_axis_name="s")

    @functools.partial(
        pl.kernel, mesh=mesh,
        out_type=jax.ShapeDtypeStruct((n_rows, HIDDEN), dtype),
        scratch_types=[
            pltpu.VMEM((ch,), jnp.int32),
            pltpu.VMEM((ch, HIDDEN), dtype),
            pltpu.SemaphoreType.DMA,
        ],
    )
    def k(table_hbm, idx_hbm, out_hbm, idx_v, rows_v, sem):
        wid = (jax.lax.axis_index("s") * info.num_cores
               + jax.lax.axis_index("c"))
        base = wid * b_per_w
        for c in range(n_ch):
            off = base + c * ch
            pltpu.sync_copy(idx_hbm.at[pl.ds(off, ch)], idx_v)
            pltpu.async_copy(table_hbm.at[idx_v], rows_v, sem).wait()
            pltpu.sync_copy(rows_v, out_hbm.at[pl.ds(off, ch)])

    return k(table, idx)


def _k6_body(be_ref, nb_ref, xs_ref, weg_ref, weu_ref, wed_ref, ys_ref):
    b = pl.program_id(0)

    @pl.when(b < nb_ref[0])
    def _():
        x = xs_ref[...]
        g = _dot(x, weg_ref[0])
        u = _dot(x, weu_ref[0])
        ys_ref[...] = _dot(g * jax.lax.logistic(g) * u, wed_ref[0])


def _grouped_moe(be, nb, xs, Weg, Weu, Wed):
    grid_spec = pltpu.PrefetchScalarGridSpec(
        num_scalar_prefetch=2,
        grid=(NBLK,),
        in_specs=[
            pl.BlockSpec((BLK, HIDDEN), lambda b, be_r, nb_r: (b, 0)),
            pl.BlockSpec((1, HIDDEN, MOE_FF),
                         lambda b, be_r, nb_r: (be_r[b], 0, 0)),
            pl.BlockSpec((1, HIDDEN, MOE_FF),
                         lambda b, be_r, nb_r: (be_r[b], 0, 0)),
            pl.BlockSpec((1, MOE_FF, HIDDEN),
                         lambda b, be_r, nb_r: (be_r[b], 0, 0)),
        ],
        out_specs=pl.BlockSpec((BLK, HIDDEN), lambda b, be_r, nb_r: (b, 0)),
    )
    return pl.pallas_call(
        _k6_body,
        grid_spec=grid_spec,
        out_shape=jax.ShapeDtypeStruct((NROWS, HIDDEN), jnp.float32),
    )(be.reshape(-1), nb.reshape(-1), xs, Weg, Weu, Wed)


def _k7_body(res2_ref, sh_ref, g0_ref, g1_ref, w_ref, out_ref):
    w = w_ref[...]
    out_ref[...] = (res2_ref[...] + sh_ref[...]
                    + w[:, 0:1] * g0_ref[...] + w[:, 1:2] * g1_ref[...])


def _final(res2, shared, g0, g1, w):
    return pl.pallas_call(
        _k7_body,
        grid=(T // BT,),
        in_specs=[
            pl.BlockSpec((BT, HIDDEN), lambda i: (i, 0)),
            pl.BlockSpec((BT, HIDDEN), lambda i: (i, 0)),
            pl.BlockSpec((BT, HIDDEN), lambda i: (i, 0)),
            pl.BlockSpec((BT, HIDDEN), lambda i: (i, 0)),
            pl.BlockSpec((BT, 2), lambda i: (i, 0)),
        ],
        out_specs=pl.BlockSpec((BT, HIDDEN), lambda i: (i, 0)),
        out_shape=jax.ShapeDtypeStruct((T, HIDDEN), jnp.float32),
        compiler_params=pltpu.CompilerParams(
            dimension_semantics=("parallel",)),
    )(res2, shared, g0, g1, w)


@jax.jit
def kernel(positions, hidden_states, Wq, bq, Wk, bk, Wv, bv, Wo, ln1, ln2,
           Wr, Weg, Weu, Wed, Wsg, Wsu, Wsd, Wse):
    q, k, v = _pre_attn(positions, hidden_states, Wq, bq, Wk, bk, Wv, bv, ln1)
    q3 = q.reshape(T, N_HEADS, HEAD_DIM).transpose(1, 0, 2)
    k3 = k.reshape(T, N_KV_HEADS, HEAD_DIM).transpose(1, 0, 2)
    v3 = v.reshape(T, N_KV_HEADS, HEAD_DIM).transpose(1, 0, 2)
    attn3 = _attention(q3, k3, v3)
    attn = attn3.transpose(1, 0, 2).reshape(T, N_HEADS * HEAD_DIM)
    res2, h2, h2b = _post_attn(attn, Wo, hidden_states, ln2)
    w, dest, src, be, nb = _router(h2, Wr)
    xs = _sc_gather(h2, src.reshape(NROWS), NROWS, jnp.float32)
    shared = _shared_expert(h2, Wsg, Wsu, Wsd, Wse)
    ys = _grouped_moe(be, nb, xs, Weg, Weu, Wed)
    g = _sc_gather(ys, dest.T.reshape(2 * T), 2 * T, jnp.float32)
    return _final(res2, shared, g[:T], g[T:], w)
